# Initial kernel scaffold; baseline (speedup 1.0000x reference)
import jax
import jax.numpy as jnp
from jax.experimental import pallas as pl


def kernel(x, edge_index, W1, as1, ad1, b1, W2, as2, ad2, b2):
    raise NotImplementedError("write your pallas kernel here")


# speedup vs baseline: 1.1623x; 1.1623x over previous
import functools

import jax
import jax.numpy as jnp
from jax.experimental import pallas as pl
from jax.experimental.pallas import tpu as pltpu

_N = 10000
_F_IN = 128
_H1, _C1 = 8, 16
_NCLS = 40
_E = 320000


def _head_matrices(as1, ad1):
    hc = _H1 * _C1
    j = jnp.arange(hc)
    head_of = j // _C1
    onehot = (head_of[:, None] == jnp.arange(_H1)[None, :]).astype(jnp.float32)
    A_s = onehot * as1.reshape(hc)[:, None]
    A_d = onehot * ad1.reshape(hc)[:, None]
    E = onehot.T
    return A_s, A_d, E


def _dense1_body(x_ref, w_ref, As_ref, Ad_ref, h_ref, asrc_ref, adst_ref):
    h = jnp.dot(x_ref[...], w_ref[...], preferred_element_type=jnp.float32)
    h_ref[...] = h
    asrc_ref[...] = jnp.dot(h, As_ref[...], preferred_element_type=jnp.float32)
    adst_ref[...] = jnp.dot(h, Ad_ref[...], preferred_element_type=jnp.float32)


def _dense1(x, W1, A_s, A_d):
    n = x.shape[0]
    return pl.pallas_call(
        _dense1_body,
        out_shape=(
            jax.ShapeDtypeStruct((n, _H1 * _C1), jnp.float32),
            jax.ShapeDtypeStruct((n, _H1), jnp.float32),
            jax.ShapeDtypeStruct((n, _H1), jnp.float32),
        ),
    )(x, W1, A_s, A_d)


def _dense2_body(acc_ref, den_ref, E_ref, b1_ref, w2_ref, as_ref, ad_ref,
                 h2_ref, asrc_ref, adst_ref):
    rden = 1.0 / (den_ref[...] + 1e-16)
    scale = jnp.dot(rden, E_ref[...], preferred_element_type=jnp.float32)
    out1 = acc_ref[...] * scale + b1_ref[...]
    h1 = jnp.where(out1 > 0, out1, jnp.exp(jnp.minimum(out1, 0.0)) - 1.0)
    h2 = jnp.dot(h1, w2_ref[...], preferred_element_type=jnp.float32)
    h2_ref[...] = h2
    asrc_ref[...] = jnp.dot(h2, as_ref[...], preferred_element_type=jnp.float32)
    adst_ref[...] = jnp.dot(h2, ad_ref[...], preferred_element_type=jnp.float32)


def _dense2(acc1, den1, E, b1, W2, as2, ad2):
    n = acc1.shape[0]
    return pl.pallas_call(
        _dense2_body,
        out_shape=(
            jax.ShapeDtypeStruct((n, _NCLS), jnp.float32),
            jax.ShapeDtypeStruct((n, 1), jnp.float32),
            jax.ShapeDtypeStruct((n, 1), jnp.float32),
        ),
    )(acc1, den1, E, b1, W2, as2.T, ad2.T)


def _final_body(acc_ref, den_ref, b2_ref, out_ref, z_ref):
    z = acc_ref[...] / (den_ref[...] + 1e-16) + b2_ref[...]
    z_ref[...] = z
    m = jnp.max(z, axis=-1, keepdims=True)
    s = jnp.log(jnp.sum(jnp.exp(z - m), axis=-1, keepdims=True))
    out_ref[...] = z - m - s


def _final(acc2, den2, b2):
    n = acc2.shape[0]
    return pl.pallas_call(
        _final_body,
        out_shape=(
            jax.ShapeDtypeStruct((n, _NCLS), jnp.float32),
            jax.ShapeDtypeStruct((n, _NCLS), jnp.float32),
        ),
    )(acc2, den2, b2)


def _edge_phase(h, asrc, adst, src, dst, n):
    heads = asrc.shape[1]
    a = asrc[src] + adst[dst]
    a = jnp.where(a > 0, a, 0.2 * a)
    e = jnp.exp(a)
    den = jax.ops.segment_sum(e, dst, num_segments=n)
    hh = h.reshape(n, heads, -1)
    msg = hh[src] * e[:, :, None]
    acc = jax.ops.segment_sum(msg, dst, num_segments=n).reshape(n, -1)
    return acc, den


def kernel(x, edge_index, W1, as1, ad1, b1, W2, as2, ad2, b2):
    n = x.shape[0]
    loop = jnp.arange(n, dtype=edge_index.dtype)
    src = jnp.concatenate([edge_index[0], loop])
    dst = jnp.concatenate([edge_index[1], loop])
    A_s, A_d, E = _head_matrices(as1, ad1)

    h1, asrc1, adst1 = _dense1(x, W1, A_s, A_d)
    acc1, den1 = _edge_phase(h1, asrc1, adst1, src, dst, n)
    h2, asrc2, adst2 = _dense2(acc1, den1, E, b1, W2, as2, ad2)
    acc2, den2 = _edge_phase(h2, asrc2, adst2, src, dst, n)
    out, z = _final(acc2, den2, b2)
    return (out, z)


# speedup vs baseline: 39.3922x; 33.8917x over previous
import functools

import jax
import jax.numpy as jnp
from jax import lax
from jax.experimental import pallas as pl
from jax.experimental.pallas import tpu as pltpu
from jax.experimental.pallas import tpu_sc as plsc

_N = 10000
_F_IN = 128
_H1, _C1 = 8, 16
_NCLS = 40
_E = 320000

_NP = 10016
_ET = 10560
_EPAD = 32 * _ET
_B1 = 480
_B2 = 480
_C2P = 48


def _head_matrices(as1, ad1):
    hc = _H1 * _C1
    j = jnp.arange(hc)
    onehot = (j[:, None] // _C1 == jnp.arange(_H1)[None, :]).astype(jnp.float32)
    A_s = onehot * as1.reshape(hc)[:, None]
    A_d = onehot * ad1.reshape(hc)[:, None]
    return A_s, A_d, onehot.T


def _dense1_body(x_ref, w_ref, As_ref, Ad_ref, h_ref, asrc_ref, adst_ref):
    h = jnp.dot(x_ref[...], w_ref[...], preferred_element_type=jnp.float32)
    h_ref[...] = h
    asrc_ref[...] = jnp.dot(h, As_ref[...], preferred_element_type=jnp.float32)
    adst_ref[...] = jnp.dot(h, Ad_ref[...], preferred_element_type=jnp.float32)


def _dense1(x, W1, A_s, A_d):
    n = x.shape[0]
    return pl.pallas_call(
        _dense1_body,
        out_shape=(
            jax.ShapeDtypeStruct((n, _H1 * _C1), jnp.float32),
            jax.ShapeDtypeStruct((n, _H1), jnp.float32),
            jax.ShapeDtypeStruct((n, _H1), jnp.float32),
        ),
    )(x, W1, A_s, A_d)


def _dense2_body(acc_ref, den_ref, E_ref, b1_ref, w2_ref, as_ref, ad_ref,
                 h2_ref, asrc_ref, adst_ref):
    rden = 1.0 / (den_ref[...] + 1e-16)
    scale = jnp.dot(rden, E_ref[...], preferred_element_type=jnp.float32)
    out1 = acc_ref[...] * scale + b1_ref[...]
    h1 = jnp.where(out1 > 0, out1, jnp.exp(jnp.minimum(out1, 0.0)) - 1.0)
    h2 = jnp.dot(h1, w2_ref[...], preferred_element_type=jnp.float32)
    h2_ref[...] = h2
    asrc_ref[...] = jnp.dot(h2, as_ref[...], preferred_element_type=jnp.float32)
    adst_ref[...] = jnp.dot(h2, ad_ref[...], preferred_element_type=jnp.float32)


def _dense2(acc1, den1, E, b1, W2, as2, ad2):
    n = acc1.shape[0]
    return pl.pallas_call(
        _dense2_body,
        out_shape=(
            jax.ShapeDtypeStruct((n, _NCLS), jnp.float32),
            jax.ShapeDtypeStruct((n, 1), jnp.float32),
            jax.ShapeDtypeStruct((n, 1), jnp.float32),
        ),
    )(acc1, den1, E, b1, W2, as2.T, ad2.T)


def _final_body(acc_ref, den_ref, b2_ref, out_ref, z_ref):
    z = acc_ref[...] / (den_ref[...] + 1e-16) + b2_ref[...]
    z_ref[...] = z
    m = jnp.max(z, axis=-1, keepdims=True)
    s = jnp.log(jnp.sum(jnp.exp(z - m), axis=-1, keepdims=True))
    out_ref[...] = z - m - s


def _final(acc2, den2, b2):
    n = acc2.shape[0]
    return pl.pallas_call(
        _final_body,
        out_shape=(
            jax.ShapeDtypeStruct((n, _NCLS), jnp.float32),
            jax.ShapeDtypeStruct((n, _NCLS), jnp.float32),
        ),
    )(acc2, den2, b2)


def _make_edge_kernel(heads, cdim, b):
    np_ = _NP
    chunks = _ET // b
    rows_total = heads * np_
    rpt = rows_total // 16
    ncopies = -(-rpt // b)

    mesh = plsc.VectorSubcoreMesh(core_axis_name="c", subcore_axis_name="s")

    @functools.partial(
        pl.kernel,
        out_type=(
            jax.ShapeDtypeStruct((2, rows_total, cdim), jnp.float32),
            jax.ShapeDtypeStruct((32, heads, np_), jnp.float32),
        ),
        mesh=mesh,
        compiler_params=pltpu.CompilerParams(
            needs_layout_passes=False, use_tc_tiling_on_sc=False),
        scratch_types=[
            pltpu.VMEM((np_,), jnp.float32),
            pltpu.VMEM((np_,), jnp.float32),
            pltpu.VMEM((np_,), jnp.float32),
            pltpu.VMEM((b,), jnp.int32),
            pltpu.VMEM((b,), jnp.int32),
            pltpu.VMEM((b,), jnp.int32),
            pltpu.VMEM((b,), jnp.int32),
            pltpu.VMEM((b,), jnp.float32),
            pltpu.VMEM((b, cdim), jnp.float32),
            pltpu.VMEM((b, cdim), jnp.float32),
            pltpu.VMEM_SHARED((rows_total, cdim), jnp.float32),
            pltpu.SemaphoreType.DMA,
        ],
    )
    def edge_kernel(h_hbm, asrcT_hbm, adstT_hbm, src_hbm, dst_hbm,
                    acc_out, den_out,
                    asrc_v, adst_v, den_v, src_v, dst_v, idxs_v, idxd_v,
                    e_buf, rows_v, msg_v, acc_sh, sem):
        cid = lax.axis_index("c")
        sid = lax.axis_index("s")
        wid = cid * 16 + sid

        def zrow(i, _):
            for kk in range(cdim // 16):
                msg_v[i, pl.ds(kk * 16, 16)] = jnp.zeros((16,), jnp.float32)
            return 0
        lax.fori_loop(0, b, zrow, 0)
        stripe0 = sid * rpt
        for ci in range(ncopies):
            start = stripe0 + jnp.minimum(ci * b, rpt - b)
            pltpu.sync_copy(msg_v, acc_sh.at[pl.ds(start, b), :])
        plsc.subcore_barrier()

        for h in range(heads):
            pltpu.sync_copy(asrcT_hbm.at[h], asrc_v)
            pltpu.sync_copy(adstT_hbm.at[h], adst_v)

            def zden(i, _):
                den_v[pl.ds(i * 16, 16)] = jnp.zeros((16,), jnp.float32)
                return 0
            lax.fori_loop(0, np_ // 16, zden, 0)

            def chunk_body(k, _):
                base = wid * _ET + k * b
                pltpu.sync_copy(src_hbm.at[pl.ds(base, b)], src_v)
                pltpu.sync_copy(dst_hbm.at[pl.ds(base, b)], dst_v)

                def ebody(j, _):
                    sv = src_v[pl.ds(j * 16, 16)]
                    dv = dst_v[pl.ds(j * 16, 16)]
                    a = (plsc.load_gather(asrc_v, [sv])
                         + plsc.load_gather(adst_v, [dv]))
                    a = jnp.where(a > 0, a, 0.2 * a)
                    e = jnp.exp(a)
                    e_buf[pl.ds(j * 16, 16)] = e
                    plsc.addupdate_scatter(den_v, [dv], e)
                    off = jnp.int32(h * np_)
                    idxs_v[pl.ds(j * 16, 16)] = sv + off
                    idxd_v[pl.ds(j * 16, 16)] = dv + off
                    return 0
                lax.fori_loop(0, b // 16, ebody, 0)

                pltpu.async_copy(h_hbm.at[idxs_v], rows_v, sem).wait()

                def mbody(j, _):
                    ev = e_buf[pl.ds(j * 16, 16)]
                    for l in range(16):
                        i = j * 16 + l
                        w = jnp.full((16,), ev[l], jnp.float32)
                        for kk in range(cdim // 16):
                            sl = pl.ds(kk * 16, 16)
                            msg_v[i, sl] = rows_v[i, sl] * w
                    return 0
                lax.fori_loop(0, b // 16, mbody, 0)

                pltpu.sync_copy(msg_v, acc_sh.at[idxd_v], add=True)
                return 0
            lax.fori_loop(0, chunks, chunk_body, 0)

            pltpu.sync_copy(den_v, den_out.at[wid, h])

        plsc.subcore_barrier()
        for ci in range(ncopies):
            start = stripe0 + jnp.minimum(ci * b, rpt - b)
            pltpu.sync_copy(acc_sh.at[pl.ds(start, b), :],
                            acc_out.at[cid, pl.ds(start, b), :])

    return edge_kernel


_edge1 = _make_edge_kernel(_H1, _C1, _B1)
_edge2 = _make_edge_kernel(1, _C2P, _B2)


def _pad_edges(edge_index, n):
    loop = jnp.arange(n, dtype=edge_index.dtype)
    npad = _EPAD - (_E + n)
    pad_ids = (jnp.arange(npad, dtype=jnp.int32) % 16) + n
    src = jnp.concatenate([edge_index[0], loop, pad_ids])
    dst = jnp.concatenate([edge_index[1], loop, pad_ids])
    return src, dst


def kernel(x, edge_index, W1, as1, ad1, b1, W2, as2, ad2, b2):
    n = x.shape[0]
    src, dst = _pad_edges(edge_index, n)
    A_s, A_d, E = _head_matrices(as1, ad1)

    h1, asrc1, adst1 = _dense1(x, W1, A_s, A_d)
    h_heads = jnp.pad(h1.reshape(n, _H1, _C1).transpose(1, 0, 2),
                      ((0, 0), (0, _NP - n), (0, 0))).reshape(_H1 * _NP, _C1)
    asrcT = jnp.pad(asrc1.T, ((0, 0), (0, _NP - n)))
    adstT = jnp.pad(adst1.T, ((0, 0), (0, _NP - n)))
    acc_parts, den_parts = _edge1(h_heads, asrcT, adstT, src, dst)
    acc1 = (acc_parts.sum(0).reshape(_H1, _NP, _C1)[:, :n]
            .transpose(1, 0, 2).reshape(n, _H1 * _C1))
    den1 = den_parts.sum(0)[:, :n].T

    h2, asrc2, adst2 = _dense2(acc1, den1, E, b1, W2, as2, ad2)
    h2_pad = jnp.pad(h2, ((0, _NP - n), (0, _C2P - _NCLS)))
    asrc2T = jnp.pad(asrc2.T, ((0, 0), (0, _NP - n)))
    adst2T = jnp.pad(adst2.T, ((0, 0), (0, _NP - n)))
    acc2_parts, den2_parts = _edge2(h2_pad, asrc2T, adst2T, src, dst)
    acc2 = acc2_parts.sum(0)[:n, :_NCLS]
    den2 = den2_parts.sum(0)[0, :n][:, None]

    out, z = _final(acc2, den2, b2)
    return (out, z)


# speedup vs baseline: 47.9145x; 1.2163x over previous
import functools

import jax
import jax.numpy as jnp
from jax import lax
from jax.experimental import pallas as pl
from jax.experimental.pallas import tpu as pltpu
from jax.experimental.pallas import tpu_sc as plsc

_N = 10000
_F_IN = 128
_H1, _C1 = 8, 16
_NCLS = 40
_E = 320000

_NP = 10016
_ET = 10560
_EPAD = 32 * _ET
_B1 = 480
_B2 = 480
_C2P = 48


def _head_matrices(as1, ad1):
    hc = _H1 * _C1
    j = jnp.arange(hc)
    onehot = (j[:, None] // _C1 == jnp.arange(_H1)[None, :]).astype(jnp.float32)
    A_s = onehot * as1.reshape(hc)[:, None]
    A_d = onehot * ad1.reshape(hc)[:, None]
    return A_s, A_d, onehot.T


def _dense1_body(x_ref, w_ref, As_ref, Ad_ref, h_ref, asrc_ref, adst_ref):
    h = jnp.dot(x_ref[...], w_ref[...], preferred_element_type=jnp.float32)
    h_ref[...] = h
    asrc_ref[...] = jnp.dot(h, As_ref[...], preferred_element_type=jnp.float32)
    adst_ref[...] = jnp.dot(h, Ad_ref[...], preferred_element_type=jnp.float32)


def _dense1(x, W1, A_s, A_d):
    n = x.shape[0]
    return pl.pallas_call(
        _dense1_body,
        out_shape=(
            jax.ShapeDtypeStruct((n, _H1 * _C1), jnp.float32),
            jax.ShapeDtypeStruct((n, _H1), jnp.float32),
            jax.ShapeDtypeStruct((n, _H1), jnp.float32),
        ),
    )(x, W1, A_s, A_d)


def _dense2_body(acc_ref, den_ref, E_ref, b1_ref, w2_ref, as_ref, ad_ref,
                 h2_ref, asrc_ref, adst_ref):
    rden = 1.0 / (den_ref[...] + 1e-16)
    scale = jnp.dot(rden, E_ref[...], preferred_element_type=jnp.float32)
    out1 = acc_ref[...] * scale + b1_ref[...]
    h1 = jnp.where(out1 > 0, out1, jnp.exp(jnp.minimum(out1, 0.0)) - 1.0)
    h2 = jnp.dot(h1, w2_ref[...], preferred_element_type=jnp.float32)
    h2_ref[...] = h2
    asrc_ref[...] = jnp.dot(h2, as_ref[...], preferred_element_type=jnp.float32)
    adst_ref[...] = jnp.dot(h2, ad_ref[...], preferred_element_type=jnp.float32)


def _dense2(acc1, den1, E, b1, W2, as2, ad2):
    n = acc1.shape[0]
    return pl.pallas_call(
        _dense2_body,
        out_shape=(
            jax.ShapeDtypeStruct((n, _NCLS), jnp.float32),
            jax.ShapeDtypeStruct((n, 1), jnp.float32),
            jax.ShapeDtypeStruct((n, 1), jnp.float32),
        ),
    )(acc1, den1, E, b1, W2, as2.T, ad2.T)


def _final_body(acc_ref, den_ref, b2_ref, out_ref, z_ref):
    z = acc_ref[...] / (den_ref[...] + 1e-16) + b2_ref[...]
    z_ref[...] = z
    m = jnp.max(z, axis=-1, keepdims=True)
    s = jnp.log(jnp.sum(jnp.exp(z - m), axis=-1, keepdims=True))
    out_ref[...] = z - m - s


def _final(acc2, den2, b2):
    n = acc2.shape[0]
    return pl.pallas_call(
        _final_body,
        out_shape=(
            jax.ShapeDtypeStruct((n, _NCLS), jnp.float32),
            jax.ShapeDtypeStruct((n, _NCLS), jnp.float32),
        ),
    )(acc2, den2, b2)


def _make_edge_kernel(heads, cdim, b):
    np_ = _NP
    K = _ET // b
    assert K % 2 == 0
    rows_total = heads * np_
    rpt = rows_total // 16
    ncopies = -(-rpt // b)

    mesh = plsc.VectorSubcoreMesh(core_axis_name="c", subcore_axis_name="s")

    edge_bufs = [
        pltpu.VMEM((b,), jnp.int32),
        pltpu.VMEM((b,), jnp.int32),
        pltpu.VMEM((b,), jnp.int32),
        pltpu.VMEM((b,), jnp.int32),
        pltpu.VMEM((b,), jnp.float32),
        pltpu.VMEM((b, cdim), jnp.float32),
    ]

    @functools.partial(
        pl.kernel,
        out_type=(
            jax.ShapeDtypeStruct((2, rows_total, cdim), jnp.float32),
            jax.ShapeDtypeStruct((32, heads, np_), jnp.float32),
        ),
        mesh=mesh,
        compiler_params=pltpu.CompilerParams(
            needs_layout_passes=False, use_tc_tiling_on_sc=False),
        scratch_types=[
            pltpu.VMEM((np_,), jnp.float32),
            pltpu.VMEM((np_,), jnp.float32),
            pltpu.VMEM((np_,), jnp.float32),
            pltpu.VMEM_SHARED((rows_total, cdim), jnp.float32),
        ] + edge_bufs + edge_bufs + [pltpu.SemaphoreType.DMA] * 8,
    )
    def edge_kernel(h_hbm, asrcT_hbm, adstT_hbm, src_hbm, dst_hbm,
                    acc_out, den_out,
                    asrc_v, adst_v, den_v, acc_sh, *bufs_and_sems):
        srcv = (bufs_and_sems[0], bufs_and_sems[6])
        dstv = (bufs_and_sems[1], bufs_and_sems[7])
        idxs = (bufs_and_sems[2], bufs_and_sems[8])
        idxd = (bufs_and_sems[3], bufs_and_sems[9])
        ebuf = (bufs_and_sems[4], bufs_and_sems[10])
        rows = (bufs_and_sems[5], bufs_and_sems[11])
        sis = (bufs_and_sems[12], bufs_and_sems[13])
        sid_ = (bufs_and_sems[14], bufs_and_sems[15])
        sg = (bufs_and_sems[16], bufs_and_sems[17])
        ss = (bufs_and_sems[18], bufs_and_sems[19])

        cid = lax.axis_index("c")
        sid = lax.axis_index("s")
        wid = cid * 16 + sid

        def stream_idx(k, p):
            base = wid * _ET + k * b
            pltpu.async_copy(src_hbm.at[pl.ds(base, b)], srcv[p], sis[p])
            pltpu.async_copy(dst_hbm.at[pl.ds(base, b)], dstv[p], sid_[p])

        def wait_idx(p):
            pltpu.make_async_copy(src_hbm.at[pl.ds(0, b)], srcv[p], sis[p]).wait()
            pltpu.make_async_copy(dst_hbm.at[pl.ds(0, b)], dstv[p], sid_[p]).wait()

        def eloop(h, p):
            def ebody(j, _):
                sv = srcv[p][pl.ds(j * 16, 16)]
                dv = dstv[p][pl.ds(j * 16, 16)]
                a = (plsc.load_gather(asrc_v, [sv])
                     + plsc.load_gather(adst_v, [dv]))
                a = jnp.where(a > 0, a, 0.2 * a)
                e = jnp.exp(a)
                ebuf[p][pl.ds(j * 16, 16)] = e
                plsc.addupdate_scatter(den_v, [dv], e)
                off = jnp.int32(h * np_)
                idxs[p][pl.ds(j * 16, 16)] = sv + off
                idxd[p][pl.ds(j * 16, 16)] = dv + off
                return 0
            lax.fori_loop(0, b // 16, ebody, 0)

        def start_gather(p):
            pltpu.async_copy(h_hbm.at[idxs[p]], rows[p], sg[p])

        def wait_gather(p):
            pltpu.make_async_copy(h_hbm.at[idxs[p]], rows[p], sg[p]).wait()

        def mloop(p):
            def mbody(j, _):
                ev = ebuf[p][pl.ds(j * 16, 16)]
                for l in range(16):
                    i = j * 16 + l
                    w = jnp.full((16,), ev[l], jnp.float32)
                    for kk in range(cdim // 16):
                        sl = pl.ds(kk * 16, 16)
                        rows[p][i, sl] = rows[p][i, sl] * w
                return 0
            lax.fori_loop(0, b // 16, mbody, 0)

        def start_scatter(p):
            pltpu.async_copy(rows[p], acc_sh.at[idxd[p]], ss[p], add=True)

        def wait_scatter(p):
            pltpu.make_async_copy(rows[p], acc_sh.at[idxd[p]], ss[p]).wait()

        def zrow(i, _):
            for kk in range(cdim // 16):
                rows[0][i, pl.ds(kk * 16, 16)] = jnp.zeros((16,), jnp.float32)
            return 0
        lax.fori_loop(0, b, zrow, 0)
        stripe0 = sid * rpt
        for ci in range(ncopies):
            start = stripe0 + jnp.minimum(ci * b, rpt - b)
            pltpu.sync_copy(rows[0], acc_sh.at[pl.ds(start, b), :])
        plsc.subcore_barrier()

        for h in range(heads):
            pltpu.sync_copy(asrcT_hbm.at[h], asrc_v)
            pltpu.sync_copy(adstT_hbm.at[h], adst_v)

            def zden(i, _):
                den_v[pl.ds(i * 16, 16)] = jnp.zeros((16,), jnp.float32)
                return 0
            lax.fori_loop(0, np_ // 16, zden, 0)

            stream_idx(0, 0)
            wait_idx(0)
            eloop(h, 0)
            start_gather(0)

            def pair(t, _):
                k0 = 2 * t
                stream_idx(k0 + 1, 1)
                wait_gather(0)
                mloop(0)
                start_scatter(0)
                wait_idx(1)

                @pl.when(t >= 1)
                def _():
                    wait_scatter(1)
                eloop(h, 1)
                start_gather(1)

                nxt = t < (K // 2 - 1)

                @pl.when(nxt)
                def _():
                    stream_idx(k0 + 2, 0)
                wait_gather(1)
                mloop(1)
                start_scatter(1)

                @pl.when(nxt)
                def _():
                    wait_idx(0)
                    wait_scatter(0)
                    eloop(h, 0)
                    start_gather(0)
                return 0
            lax.fori_loop(0, K // 2, pair, 0)

            wait_scatter(0)
            wait_scatter(1)
            pltpu.sync_copy(den_v, den_out.at[wid, h])

        plsc.subcore_barrier()
        for ci in range(ncopies):
            start = stripe0 + jnp.minimum(ci * b, rpt - b)
            pltpu.sync_copy(acc_sh.at[pl.ds(start, b), :],
                            acc_out.at[cid, pl.ds(start, b), :])

    return edge_kernel


_edge1 = _make_edge_kernel(_H1, _C1, _B1)
_edge2 = _make_edge_kernel(1, _C2P, _B2)


def _pad_edges(edge_index, n):
    loop = jnp.arange(n, dtype=edge_index.dtype)
    npad = _EPAD - (_E + n)
    pad_ids = (jnp.arange(npad, dtype=jnp.int32) % 16) + n
    src = jnp.concatenate([edge_index[0], loop, pad_ids])
    dst = jnp.concatenate([edge_index[1], loop, pad_ids])
    return src, dst


def kernel(x, edge_index, W1, as1, ad1, b1, W2, as2, ad2, b2):
    n = x.shape[0]
    src, dst = _pad_edges(edge_index, n)
    A_s, A_d, E = _head_matrices(as1, ad1)

    h1, asrc1, adst1 = _dense1(x, W1, A_s, A_d)
    h_heads = jnp.pad(h1.reshape(n, _H1, _C1).transpose(1, 0, 2),
                      ((0, 0), (0, _NP - n), (0, 0))).reshape(_H1 * _NP, _C1)
    asrcT = jnp.pad(asrc1.T, ((0, 0), (0, _NP - n)))
    adstT = jnp.pad(adst1.T, ((0, 0), (0, _NP - n)))
    acc_parts, den_parts = _edge1(h_heads, asrcT, adstT, src, dst)
    acc1 = (acc_parts.sum(0).reshape(_H1, _NP, _C1)[:, :n]
            .transpose(1, 0, 2).reshape(n, _H1 * _C1))
    den1 = den_parts.sum(0)[:, :n].T

    h2, asrc2, adst2 = _dense2(acc1, den1, E, b1, W2, as2, ad2)
    h2_pad = jnp.pad(h2, ((0, _NP - n), (0, _C2P - _NCLS)))
    asrc2T = jnp.pad(asrc2.T, ((0, 0), (0, _NP - n)))
    adst2T = jnp.pad(adst2.T, ((0, 0), (0, _NP - n)))
    acc2_parts, den2_parts = _edge2(h2_pad, asrc2T, adst2T, src, dst)
    acc2 = acc2_parts.sum(0)[:n, :_NCLS]
    den2 = den2_parts.sum(0)[0, :n][:, None]

    out, z = _final(acc2, den2, b2)
    return (out, z)


# speedup vs baseline: 49.0349x; 1.0234x over previous
import functools

import jax
import jax.numpy as jnp
from jax import lax
from jax.experimental import pallas as pl
from jax.experimental.pallas import tpu as pltpu
from jax.experimental.pallas import tpu_sc as plsc

_N = 10000
_F_IN = 128
_H1, _C1 = 8, 16
_NCLS = 40
_E = 320000

_NP = 10016
_ET = 10560
_EPAD = 32 * _ET
_B1 = 480
_B2 = 480
_C2P = 48


def _head_matrices(as1, ad1):
    hc = _H1 * _C1
    j = jnp.arange(hc)
    onehot = (j[:, None] // _C1 == jnp.arange(_H1)[None, :]).astype(jnp.float32)
    A_s = onehot * as1.reshape(hc)[:, None]
    A_d = onehot * ad1.reshape(hc)[:, None]
    return A_s, A_d, onehot.T


def _dense1_body(x_ref, w_ref, As_ref, Ad_ref, h_ref, asrc_ref, adst_ref):
    h = jnp.dot(x_ref[...], w_ref[...], preferred_element_type=jnp.float32)
    h_ref[...] = h
    asrc_ref[...] = jnp.dot(h, As_ref[...], preferred_element_type=jnp.float32)
    adst_ref[...] = jnp.dot(h, Ad_ref[...], preferred_element_type=jnp.float32)


def _dense1(x, W1, A_s, A_d):
    n = x.shape[0]
    return pl.pallas_call(
        _dense1_body,
        out_shape=(
            jax.ShapeDtypeStruct((n, _H1 * _C1), jnp.float32),
            jax.ShapeDtypeStruct((n, _H1), jnp.float32),
            jax.ShapeDtypeStruct((n, _H1), jnp.float32),
        ),
    )(x, W1, A_s, A_d)


def _dense2_body(acc_ref, den_ref, E_ref, b1_ref, w2_ref, as_ref, ad_ref,
                 h2_ref, asrc_ref, adst_ref):
    rden = 1.0 / (den_ref[...] + 1e-16)
    scale = jnp.dot(rden, E_ref[...], preferred_element_type=jnp.float32)
    out1 = acc_ref[...] * scale + b1_ref[...]
    h1 = jnp.where(out1 > 0, out1, jnp.exp(jnp.minimum(out1, 0.0)) - 1.0)
    h2 = jnp.dot(h1, w2_ref[...], preferred_element_type=jnp.float32)
    h2_ref[...] = h2
    asrc_ref[...] = jnp.dot(h2, as_ref[...], preferred_element_type=jnp.float32)
    adst_ref[...] = jnp.dot(h2, ad_ref[...], preferred_element_type=jnp.float32)


def _dense2(acc1, den1, E, b1, W2, as2, ad2):
    n = acc1.shape[0]
    return pl.pallas_call(
        _dense2_body,
        out_shape=(
            jax.ShapeDtypeStruct((n, _NCLS), jnp.float32),
            jax.ShapeDtypeStruct((n, 1), jnp.float32),
            jax.ShapeDtypeStruct((n, 1), jnp.float32),
        ),
    )(acc1, den1, E, b1, W2, as2.T, ad2.T)


def _final_body(acc_ref, den_ref, b2_ref, out_ref, z_ref):
    z = acc_ref[...] / (den_ref[...] + 1e-16) + b2_ref[...]
    z_ref[...] = z
    m = jnp.max(z, axis=-1, keepdims=True)
    s = jnp.log(jnp.sum(jnp.exp(z - m), axis=-1, keepdims=True))
    out_ref[...] = z - m - s


def _final(acc2, den2, b2):
    n = acc2.shape[0]
    return pl.pallas_call(
        _final_body,
        out_shape=(
            jax.ShapeDtypeStruct((n, _NCLS), jnp.float32),
            jax.ShapeDtypeStruct((n, _NCLS), jnp.float32),
        ),
    )(acc2, den2, b2)


def _make_edge_kernel(heads, cdim, b):
    np_ = _NP
    K = _ET // b
    assert K % 2 == 0
    rows_total = heads * np_
    rpt = rows_total // 16
    ncopies = -(-rpt // b)

    mesh = plsc.VectorSubcoreMesh(core_axis_name="c", subcore_axis_name="s")

    edge_bufs = [
        pltpu.VMEM((b,), jnp.int32),
        pltpu.VMEM((b,), jnp.int32),
        pltpu.VMEM((b,), jnp.int32),
        pltpu.VMEM((b,), jnp.int32),
        pltpu.VMEM((b,), jnp.float32),
        pltpu.VMEM((b, cdim), jnp.float32),
    ]

    @functools.partial(
        pl.kernel,
        out_type=(
            jax.ShapeDtypeStruct((2, rows_total, cdim), jnp.float32),
            jax.ShapeDtypeStruct((32, heads, np_), jnp.float32),
        ),
        mesh=mesh,
        compiler_params=pltpu.CompilerParams(
            needs_layout_passes=False, use_tc_tiling_on_sc=False),
        scratch_types=[
            pltpu.VMEM((np_,), jnp.float32),
            pltpu.VMEM((np_,), jnp.float32),
            pltpu.VMEM((np_,), jnp.float32),
            pltpu.VMEM_SHARED((rows_total, cdim), jnp.float32),
        ] + edge_bufs + edge_bufs + [pltpu.SemaphoreType.DMA] * 8,
    )
    def edge_kernel(h_hbm, asrcT_hbm, adstT_hbm, src_hbm, dst_hbm,
                    acc_out, den_out,
                    asrc_v, adst_v, den_v, acc_sh, *bufs_and_sems):
        srcv = (bufs_and_sems[0], bufs_and_sems[6])
        dstv = (bufs_and_sems[1], bufs_and_sems[7])
        idxs = (bufs_and_sems[2], bufs_and_sems[8])
        idxd = (bufs_and_sems[3], bufs_and_sems[9])
        ebuf = (bufs_and_sems[4], bufs_and_sems[10])
        rows = (bufs_and_sems[5], bufs_and_sems[11])
        sis = (bufs_and_sems[12], bufs_and_sems[13])
        sid_ = (bufs_and_sems[14], bufs_and_sems[15])
        sg = (bufs_and_sems[16], bufs_and_sems[17])
        ss = (bufs_and_sems[18], bufs_and_sems[19])

        cid = lax.axis_index("c")
        sid = lax.axis_index("s")
        wid = cid * 16 + sid

        def stream_idx(k, p):
            base = wid * _ET + k * b
            pltpu.async_copy(src_hbm.at[pl.ds(base, b)], srcv[p], sis[p])
            pltpu.async_copy(dst_hbm.at[pl.ds(base, b)], dstv[p], sid_[p])

        def wait_idx(p):
            pltpu.make_async_copy(src_hbm.at[pl.ds(0, b)], srcv[p], sis[p]).wait()
            pltpu.make_async_copy(dst_hbm.at[pl.ds(0, b)], dstv[p], sid_[p]).wait()

        def eloop(h, p):
            def ebody(j, _):
                sv = srcv[p][pl.ds(j * 16, 16)]
                dv = dstv[p][pl.ds(j * 16, 16)]
                a = (plsc.load_gather(asrc_v, [sv])
                     + plsc.load_gather(adst_v, [dv]))
                a = jnp.where(a > 0, a, 0.2 * a)
                e = jnp.exp(a)
                ebuf[p][pl.ds(j * 16, 16)] = e
                plsc.addupdate_scatter(den_v, [dv], e)
                off = jnp.int32(h * np_)
                idxs[p][pl.ds(j * 16, 16)] = sv + off
                idxd[p][pl.ds(j * 16, 16)] = dv + off
                return 0
            lax.fori_loop(0, b // 16, ebody, 0)

        def start_gather(p):
            pltpu.async_copy(h_hbm.at[idxs[p]], rows[p], sg[p])

        def wait_gather(p):
            pltpu.make_async_copy(h_hbm.at[idxs[p]], rows[p], sg[p]).wait()

        def mloop(p):
            @plsc.parallel_loop(0, b // 16, unroll=2)
            def mbody(j):
                ev = ebuf[p][pl.ds(j * 16, 16)]
                for l in range(16):
                    i = j * 16 + l
                    w = jnp.full((16,), ev[l], jnp.float32)
                    for kk in range(cdim // 16):
                        sl = pl.ds(kk * 16, 16)
                        rows[p][i, sl] = rows[p][i, sl] * w

        def start_scatter(p):
            pltpu.async_copy(rows[p], acc_sh.at[idxd[p]], ss[p], add=True)

        def wait_scatter(p):
            pltpu.make_async_copy(rows[p], acc_sh.at[idxd[p]], ss[p]).wait()

        def zrow(i, _):
            for kk in range(cdim // 16):
                rows[0][i, pl.ds(kk * 16, 16)] = jnp.zeros((16,), jnp.float32)
            return 0
        lax.fori_loop(0, b, zrow, 0)
        stripe0 = sid * rpt
        for ci in range(ncopies):
            start = stripe0 + jnp.minimum(ci * b, rpt - b)
            pltpu.sync_copy(rows[0], acc_sh.at[pl.ds(start, b), :])
        plsc.subcore_barrier()

        for h in range(heads):
            pltpu.sync_copy(asrcT_hbm.at[h], asrc_v)
            pltpu.sync_copy(adstT_hbm.at[h], adst_v)

            def zden(i, _):
                den_v[pl.ds(i * 16, 16)] = jnp.zeros((16,), jnp.float32)
                return 0
            lax.fori_loop(0, np_ // 16, zden, 0)

            stream_idx(0, 0)
            wait_idx(0)
            eloop(h, 0)
            start_gather(0)

            def pair(t, _):
                k0 = 2 * t
                stream_idx(k0 + 1, 1)
                wait_gather(0)
                mloop(0)
                start_scatter(0)
                wait_idx(1)

                @pl.when(t >= 1)
                def _():
                    wait_scatter(1)
                eloop(h, 1)
                start_gather(1)

                nxt = t < (K // 2 - 1)

                @pl.when(nxt)
                def _():
                    stream_idx(k0 + 2, 0)
                wait_gather(1)
                mloop(1)
                start_scatter(1)

                @pl.when(nxt)
                def _():
                    wait_idx(0)
                    wait_scatter(0)
                    eloop(h, 0)
                    start_gather(0)
                return 0
            lax.fori_loop(0, K // 2, pair, 0)

            wait_scatter(0)
            wait_scatter(1)
            pltpu.sync_copy(den_v, den_out.at[wid, h])

        plsc.subcore_barrier()
        for ci in range(ncopies):
            start = stripe0 + jnp.minimum(ci * b, rpt - b)
            pltpu.sync_copy(acc_sh.at[pl.ds(start, b), :],
                            acc_out.at[cid, pl.ds(start, b), :])

    return edge_kernel


_edge1 = _make_edge_kernel(_H1, _C1, _B1)
_edge2 = _make_edge_kernel(1, _C2P, _B2)


def _pad_edges(edge_index, n):
    loop = jnp.arange(n, dtype=edge_index.dtype)
    npad = _EPAD - (_E + n)
    pad_ids = (jnp.arange(npad, dtype=jnp.int32) % 16) + n
    src = jnp.concatenate([edge_index[0], loop, pad_ids])
    dst = jnp.concatenate([edge_index[1], loop, pad_ids])
    return src, dst


def kernel(x, edge_index, W1, as1, ad1, b1, W2, as2, ad2, b2):
    n = x.shape[0]
    src, dst = _pad_edges(edge_index, n)
    A_s, A_d, E = _head_matrices(as1, ad1)

    h1, asrc1, adst1 = _dense1(x, W1, A_s, A_d)
    h_heads = jnp.concatenate(
        [jnp.pad(h1[:, _C1 * h:_C1 * (h + 1)], ((0, _NP - n), (0, 0)))
         for h in range(_H1)], axis=0)
    asrcT = jnp.pad(asrc1.T, ((0, 0), (0, _NP - n)))
    adstT = jnp.pad(adst1.T, ((0, 0), (0, _NP - n)))
    acc_parts, den_parts = _edge1(h_heads, asrcT, adstT, src, dst)
    acc_sum = acc_parts.sum(0)
    acc1 = jnp.concatenate(
        [acc_sum[_NP * h:_NP * h + n, :] for h in range(_H1)], axis=1)
    den1 = den_parts.sum(0)[:, :n].T

    h2, asrc2, adst2 = _dense2(acc1, den1, E, b1, W2, as2, ad2)
    h2_pad = jnp.pad(h2, ((0, _NP - n), (0, _C2P - _NCLS)))
    asrc2T = jnp.pad(asrc2.T, ((0, 0), (0, _NP - n)))
    adst2T = jnp.pad(adst2.T, ((0, 0), (0, _NP - n)))
    acc2_parts, den2_parts = _edge2(h2_pad, asrc2T, adst2T, src, dst)
    acc2 = acc2_parts.sum(0)[:n, :_NCLS]
    den2 = den2_parts.sum(0)[0, :n][:, None]

    out, z = _final(acc2, den2, b2)
    return (out, z)


# speedup vs baseline: 51.9365x; 1.0592x over previous
import functools

import jax
import jax.numpy as jnp
from jax import lax
from jax.experimental import pallas as pl
from jax.experimental.pallas import tpu as pltpu
from jax.experimental.pallas import tpu_sc as plsc

_N = 10000
_F_IN = 128
_H1, _C1 = 8, 16
_NCLS = 40
_E = 320000

_NP = 10016
_ET = 10560
_EPAD = 32 * _ET
_B1 = 480
_B2 = 480
_C2P = 48


def _head_matrices(as1, ad1):
    hc = _H1 * _C1
    j = jnp.arange(hc)
    onehot = (j[:, None] // _C1 == jnp.arange(_H1)[None, :]).astype(jnp.float32)
    A_s = onehot * as1.reshape(hc)[:, None]
    A_d = onehot * ad1.reshape(hc)[:, None]
    return A_s, A_d, onehot.T


def _dense1_body(x_ref, w_ref, As_ref, Ad_ref, h_ref, asrc_ref, adst_ref):
    h = jnp.dot(x_ref[...], w_ref[...], preferred_element_type=jnp.float32)
    h_ref[...] = h
    asrc_ref[...] = jnp.dot(h, As_ref[...], preferred_element_type=jnp.float32)
    adst_ref[...] = jnp.dot(h, Ad_ref[...], preferred_element_type=jnp.float32)


def _dense1(x, W1, A_s, A_d):
    n = x.shape[0]
    return pl.pallas_call(
        _dense1_body,
        out_shape=(
            jax.ShapeDtypeStruct((n, _H1 * _C1), jnp.float32),
            jax.ShapeDtypeStruct((n, _H1), jnp.float32),
            jax.ShapeDtypeStruct((n, _H1), jnp.float32),
        ),
    )(x, W1, A_s, A_d)


def _dense2_body(acc_ref, den_ref, E_ref, b1_ref, w2_ref, as_ref, ad_ref,
                 h2_ref, asrc_ref, adst_ref):
    rden = 1.0 / (den_ref[...] + 1e-16)
    scale = jnp.dot(rden, E_ref[...], preferred_element_type=jnp.float32)
    out1 = acc_ref[...] * scale + b1_ref[...]
    h1 = jnp.where(out1 > 0, out1, jnp.exp(jnp.minimum(out1, 0.0)) - 1.0)
    h2 = jnp.dot(h1, w2_ref[...], preferred_element_type=jnp.float32)
    h2_ref[...] = h2
    asrc_ref[...] = jnp.dot(h2, as_ref[...], preferred_element_type=jnp.float32)
    adst_ref[...] = jnp.dot(h2, ad_ref[...], preferred_element_type=jnp.float32)


def _dense2(acc1, den1, E, b1, W2, as2, ad2):
    n = acc1.shape[0]
    return pl.pallas_call(
        _dense2_body,
        out_shape=(
            jax.ShapeDtypeStruct((n, _NCLS), jnp.float32),
            jax.ShapeDtypeStruct((n, 1), jnp.float32),
            jax.ShapeDtypeStruct((n, 1), jnp.float32),
        ),
    )(acc1, den1, E, b1, W2, as2.T, ad2.T)


def _final_body(acc_ref, den_ref, b2_ref, out_ref, z_ref):
    z = acc_ref[...] / (den_ref[...] + 1e-16) + b2_ref[...]
    z_ref[...] = z
    m = jnp.max(z, axis=-1, keepdims=True)
    s = jnp.log(jnp.sum(jnp.exp(z - m), axis=-1, keepdims=True))
    out_ref[...] = z - m - s


def _final(acc2, den2, b2):
    n = acc2.shape[0]
    return pl.pallas_call(
        _final_body,
        out_shape=(
            jax.ShapeDtypeStruct((n, _NCLS), jnp.float32),
            jax.ShapeDtypeStruct((n, _NCLS), jnp.float32),
        ),
    )(acc2, den2, b2)


def _make_edge_kernel(heads, cdim, b):
    np_ = _NP
    K = _ET // b
    assert K % 2 == 0
    rows_total = heads * np_
    rpt = rows_total // 16
    ncopies = -(-rpt // b)

    mesh = plsc.VectorSubcoreMesh(core_axis_name="c", subcore_axis_name="s")

    edge_bufs = [
        pltpu.VMEM((b,), jnp.int32),
        pltpu.VMEM((b,), jnp.int32),
        pltpu.VMEM((b,), jnp.int32),
        pltpu.VMEM((b,), jnp.int32),
        pltpu.VMEM((b,), jnp.float32),
        pltpu.VMEM((b, cdim), jnp.float32),
    ]

    @functools.partial(
        pl.kernel,
        out_type=(
            jax.ShapeDtypeStruct((2, rows_total, cdim), jnp.float32),
            jax.ShapeDtypeStruct((32, heads, np_), jnp.float32),
        ),
        mesh=mesh,
        compiler_params=pltpu.CompilerParams(
            needs_layout_passes=False, use_tc_tiling_on_sc=False),
        scratch_types=[
            pltpu.VMEM((np_,), jnp.float32),
            pltpu.VMEM((np_,), jnp.float32),
            pltpu.VMEM((np_,), jnp.float32),
            pltpu.VMEM_SHARED((rows_total, cdim), jnp.float32),
        ] + edge_bufs + edge_bufs + [pltpu.SemaphoreType.DMA] * 8,
    )
    def edge_kernel(h_hbm, asrcT_hbm, adstT_hbm, src_hbm, dst_hbm,
                    acc_out, den_out,
                    asrc_v, adst_v, den_v, acc_sh, *bufs_and_sems):
        srcv = (bufs_and_sems[0], bufs_and_sems[6])
        dstv = (bufs_and_sems[1], bufs_and_sems[7])
        idxs = (bufs_and_sems[2], bufs_and_sems[8])
        idxd = (bufs_and_sems[3], bufs_and_sems[9])
        ebuf = (bufs_and_sems[4], bufs_and_sems[10])
        rows = (bufs_and_sems[5], bufs_and_sems[11])
        sis = (bufs_and_sems[12], bufs_and_sems[13])
        sid_ = (bufs_and_sems[14], bufs_and_sems[15])
        sg = (bufs_and_sems[16], bufs_and_sems[17])
        ss = (bufs_and_sems[18], bufs_and_sems[19])

        cid = lax.axis_index("c")
        sid = lax.axis_index("s")
        wid = cid * 16 + sid

        def stream_idx(k, p):
            base = wid * _ET + k * b
            pltpu.async_copy(src_hbm.at[pl.ds(base, b)], srcv[p], sis[p])
            pltpu.async_copy(dst_hbm.at[pl.ds(base, b)], dstv[p], sid_[p])

        def wait_idx(p):
            pltpu.make_async_copy(src_hbm.at[pl.ds(0, b)], srcv[p], sis[p]).wait()
            pltpu.make_async_copy(dst_hbm.at[pl.ds(0, b)], dstv[p], sid_[p]).wait()

        def eloop(h, p):
            @plsc.parallel_loop(0, b // 16, unroll=2)
            def ebody(j):
                sv = srcv[p][pl.ds(j * 16, 16)]
                dv = dstv[p][pl.ds(j * 16, 16)]
                a = (plsc.load_gather(asrc_v, [sv])
                     + plsc.load_gather(adst_v, [dv]))
                a = jnp.where(a > 0, a, 0.2 * a)
                e = jnp.exp(a)
                ebuf[p][pl.ds(j * 16, 16)] = e
                off = jnp.int32(h * np_)
                idxs[p][pl.ds(j * 16, 16)] = sv + off
                idxd[p][pl.ds(j * 16, 16)] = dv + off

            def dbody(j, _):
                dv = dstv[p][pl.ds(j * 16, 16)]
                e = ebuf[p][pl.ds(j * 16, 16)]
                plsc.addupdate_scatter(den_v, [dv], e)
                return 0
            lax.fori_loop(0, b // 16, dbody, 0)

        def start_gather(p):
            pltpu.async_copy(h_hbm.at[idxs[p]], rows[p], sg[p])

        def wait_gather(p):
            pltpu.make_async_copy(h_hbm.at[idxs[p]], rows[p], sg[p]).wait()

        def mloop(p):
            @plsc.parallel_loop(0, b // 16, unroll=2)
            def mbody(j):
                ev = ebuf[p][pl.ds(j * 16, 16)]
                for l in range(16):
                    i = j * 16 + l
                    w = jnp.full((16,), ev[l], jnp.float32)
                    for kk in range(cdim // 16):
                        sl = pl.ds(kk * 16, 16)
                        rows[p][i, sl] = rows[p][i, sl] * w

        def start_scatter(p):
            pltpu.async_copy(rows[p], acc_sh.at[idxd[p]], ss[p], add=True)

        def wait_scatter(p):
            pltpu.make_async_copy(rows[p], acc_sh.at[idxd[p]], ss[p]).wait()

        def zrow(i, _):
            for kk in range(cdim // 16):
                rows[0][i, pl.ds(kk * 16, 16)] = jnp.zeros((16,), jnp.float32)
            return 0
        lax.fori_loop(0, b, zrow, 0)
        stripe0 = sid * rpt
        for ci in range(ncopies):
            start = stripe0 + jnp.minimum(ci * b, rpt - b)
            pltpu.sync_copy(rows[0], acc_sh.at[pl.ds(start, b), :])
        plsc.subcore_barrier()

        for h in range(heads):
            pltpu.sync_copy(asrcT_hbm.at[h], asrc_v)
            pltpu.sync_copy(adstT_hbm.at[h], adst_v)

            def zden(i, _):
                den_v[pl.ds(i * 16, 16)] = jnp.zeros((16,), jnp.float32)
                return 0
            lax.fori_loop(0, np_ // 16, zden, 0)

            stream_idx(0, 0)
            wait_idx(0)
            eloop(h, 0)
            start_gather(0)

            def pair(t, _):
                k0 = 2 * t
                stream_idx(k0 + 1, 1)
                wait_gather(0)
                mloop(0)
                start_scatter(0)
                wait_idx(1)

                @pl.when(t >= 1)
                def _():
                    wait_scatter(1)
                eloop(h, 1)
                start_gather(1)

                nxt = t < (K // 2 - 1)

                @pl.when(nxt)
                def _():
                    stream_idx(k0 + 2, 0)
                wait_gather(1)
                mloop(1)
                start_scatter(1)

                @pl.when(nxt)
                def _():
                    wait_idx(0)
                    wait_scatter(0)
                    eloop(h, 0)
                    start_gather(0)
                return 0
            lax.fori_loop(0, K // 2, pair, 0)

            wait_scatter(0)
            wait_scatter(1)
            pltpu.sync_copy(den_v, den_out.at[wid, h])

        plsc.subcore_barrier()
        for ci in range(ncopies):
            start = stripe0 + jnp.minimum(ci * b, rpt - b)
            pltpu.sync_copy(acc_sh.at[pl.ds(start, b), :],
                            acc_out.at[cid, pl.ds(start, b), :])

    return edge_kernel


_edge1 = _make_edge_kernel(_H1, _C1, _B1)
_edge2 = _make_edge_kernel(1, _C2P, _B2)


def _pad_edges(edge_index, n):
    loop = jnp.arange(n, dtype=edge_index.dtype)
    npad = _EPAD - (_E + n)
    pad_ids = (jnp.arange(npad, dtype=jnp.int32) % 16) + n
    src = jnp.concatenate([edge_index[0], loop, pad_ids])
    dst = jnp.concatenate([edge_index[1], loop, pad_ids])
    return src, dst


def kernel(x, edge_index, W1, as1, ad1, b1, W2, as2, ad2, b2):
    n = x.shape[0]
    src, dst = _pad_edges(edge_index, n)
    A_s, A_d, E = _head_matrices(as1, ad1)

    h1, asrc1, adst1 = _dense1(x, W1, A_s, A_d)
    h_heads = jnp.concatenate(
        [jnp.pad(h1[:, _C1 * h:_C1 * (h + 1)], ((0, _NP - n), (0, 0)))
         for h in range(_H1)], axis=0)
    asrcT = jnp.pad(asrc1.T, ((0, 0), (0, _NP - n)))
    adstT = jnp.pad(adst1.T, ((0, 0), (0, _NP - n)))
    acc_parts, den_parts = _edge1(h_heads, asrcT, adstT, src, dst)
    acc_sum = acc_parts.sum(0)
    acc1 = jnp.concatenate(
        [acc_sum[_NP * h:_NP * h + n, :] for h in range(_H1)], axis=1)
    den1 = den_parts.sum(0)[:, :n].T

    h2, asrc2, adst2 = _dense2(acc1, den1, E, b1, W2, as2, ad2)
    h2_pad = jnp.pad(h2, ((0, _NP - n), (0, _C2P - _NCLS)))
    asrc2T = jnp.pad(asrc2.T, ((0, 0), (0, _NP - n)))
    adst2T = jnp.pad(adst2.T, ((0, 0), (0, _NP - n)))
    acc2_parts, den2_parts = _edge2(h2_pad, asrc2T, adst2T, src, dst)
    acc2 = acc2_parts.sum(0)[:n, :_NCLS]
    den2 = den2_parts.sum(0)[0, :n][:, None]

    out, z = _final(acc2, den2, b2)
    return (out, z)


# speedup vs baseline: 61.4276x; 1.1827x over previous
import functools

import jax
import jax.numpy as jnp
from jax import lax
from jax.experimental import pallas as pl
from jax.experimental.pallas import tpu as pltpu
from jax.experimental.pallas import tpu_sc as plsc

_N = 10000
_F_IN = 128
_H1, _C1 = 8, 16
_NCLS = 40
_E = 320000

_NP = 10016
_ET = 10560
_EPAD = 32 * _ET
_B1 = 480
_B2 = 480
_C2P = 48


def _head_matrices(as1, ad1):
    hc = _H1 * _C1
    j = jnp.arange(hc)
    onehot = (j[:, None] // _C1 == jnp.arange(_H1)[None, :]).astype(jnp.float32)
    A_s = onehot * as1.reshape(hc)[:, None]
    A_d = onehot * ad1.reshape(hc)[:, None]
    return A_s, A_d, onehot.T


def _dense1_body(x_ref, w_ref, As_ref, Ad_ref, h_ref, asrc_ref, adst_ref):
    h = jnp.dot(x_ref[...], w_ref[...], preferred_element_type=jnp.float32)
    h_ref[...] = h
    asrc_ref[...] = jnp.dot(h, As_ref[...], preferred_element_type=jnp.float32)
    adst_ref[...] = jnp.dot(h, Ad_ref[...], preferred_element_type=jnp.float32)


def _dense1(x, W1, A_s, A_d):
    n = x.shape[0]
    return pl.pallas_call(
        _dense1_body,
        out_shape=(
            jax.ShapeDtypeStruct((n, _H1 * _C1), jnp.float32),
            jax.ShapeDtypeStruct((n, _H1), jnp.float32),
            jax.ShapeDtypeStruct((n, _H1), jnp.float32),
        ),
    )(x, W1, A_s, A_d)


def _dense2_body(acc_ref, den_ref, E_ref, b1_ref, w2_ref, as_ref, ad_ref,
                 h2_ref, asrc_ref, adst_ref):
    rden = 1.0 / (den_ref[...] + 1e-16)
    scale = jnp.dot(rden, E_ref[...], preferred_element_type=jnp.float32)
    out1 = acc_ref[...] * scale + b1_ref[...]
    h1 = jnp.where(out1 > 0, out1, jnp.exp(jnp.minimum(out1, 0.0)) - 1.0)
    h2 = jnp.dot(h1, w2_ref[...], preferred_element_type=jnp.float32)
    h2_ref[...] = h2
    asrc_ref[...] = jnp.dot(h2, as_ref[...], preferred_element_type=jnp.float32)
    adst_ref[...] = jnp.dot(h2, ad_ref[...], preferred_element_type=jnp.float32)


def _dense2(acc1, den1, E, b1, W2, as2, ad2):
    n = acc1.shape[0]
    return pl.pallas_call(
        _dense2_body,
        out_shape=(
            jax.ShapeDtypeStruct((n, _NCLS), jnp.float32),
            jax.ShapeDtypeStruct((n, 1), jnp.float32),
            jax.ShapeDtypeStruct((n, 1), jnp.float32),
        ),
    )(acc1, den1, E, b1, W2, as2.T, ad2.T)


def _final_body(acc_ref, den_ref, b2_ref, out_ref, z_ref):
    z = acc_ref[...] / (den_ref[...] + 1e-16) + b2_ref[...]
    z_ref[...] = z
    m = jnp.max(z, axis=-1, keepdims=True)
    s = jnp.log(jnp.sum(jnp.exp(z - m), axis=-1, keepdims=True))
    out_ref[...] = z - m - s


def _final(acc2, den2, b2):
    n = acc2.shape[0]
    return pl.pallas_call(
        _final_body,
        out_shape=(
            jax.ShapeDtypeStruct((n, _NCLS), jnp.float32),
            jax.ShapeDtypeStruct((n, _NCLS), jnp.float32),
        ),
    )(acc2, den2, b2)


def _make_edge_kernel(heads, cdim, b):
    np_ = _NP
    K = _ET // b
    assert K % 2 == 0
    rows_total = heads * np_
    rpt = rows_total // 16
    ncopies = -(-rpt // b)

    mesh = plsc.VectorSubcoreMesh(core_axis_name="c", subcore_axis_name="s")

    edge_bufs = [
        pltpu.VMEM((b,), jnp.int32),
        pltpu.VMEM((b,), jnp.int32),
        pltpu.VMEM((b,), jnp.int32),
        pltpu.VMEM((b,), jnp.int32),
        pltpu.VMEM((b,), jnp.float32),
        pltpu.VMEM((b, cdim), jnp.float32),
    ]

    @functools.partial(
        pl.kernel,
        out_type=(
            jax.ShapeDtypeStruct((2, rows_total, cdim), jnp.float32),
            jax.ShapeDtypeStruct((32, heads, np_), jnp.float32),
        ),
        mesh=mesh,
        compiler_params=pltpu.CompilerParams(
            needs_layout_passes=False, use_tc_tiling_on_sc=False),
        scratch_types=[
            pltpu.VMEM((np_,), jnp.float32),
            pltpu.VMEM((np_,), jnp.float32),
            pltpu.VMEM((np_,), jnp.float32),
            pltpu.VMEM_SHARED((rows_total, cdim), jnp.float32),
        ] + edge_bufs + edge_bufs + [pltpu.SemaphoreType.DMA] * 8,
    )
    def edge_kernel(h_hbm, asrcT_hbm, adstT_hbm, src_hbm, dst_hbm,
                    acc_out, den_out,
                    asrc_v, adst_v, den_v, acc_sh, *bufs_and_sems):
        srcv = (bufs_and_sems[0], bufs_and_sems[6])
        dstv = (bufs_and_sems[1], bufs_and_sems[7])
        idxs = (bufs_and_sems[2], bufs_and_sems[8])
        idxd = (bufs_and_sems[3], bufs_and_sems[9])
        ebuf = (bufs_and_sems[4], bufs_and_sems[10])
        rows = (bufs_and_sems[5], bufs_and_sems[11])
        sis = (bufs_and_sems[12], bufs_and_sems[13])
        sid_ = (bufs_and_sems[14], bufs_and_sems[15])
        sg = (bufs_and_sems[16], bufs_and_sems[17])
        ss = (bufs_and_sems[18], bufs_and_sems[19])

        cid = lax.axis_index("c")
        sid = lax.axis_index("s")
        wid = cid * 16 + sid

        def stream_idx(k, p):
            base = wid * _ET + k * b
            pltpu.async_copy(src_hbm.at[pl.ds(base, b)], srcv[p], sis[p])
            pltpu.async_copy(dst_hbm.at[pl.ds(base, b)], dstv[p], sid_[p])

        def wait_idx(p):
            pltpu.make_async_copy(src_hbm.at[pl.ds(0, b)], srcv[p], sis[p]).wait()
            pltpu.make_async_copy(dst_hbm.at[pl.ds(0, b)], dstv[p], sid_[p]).wait()

        def eloop(h, p):
            @plsc.parallel_loop(0, b // 16, unroll=2)
            def ebody(j):
                sv = srcv[p][pl.ds(j * 16, 16)]
                dv = dstv[p][pl.ds(j * 16, 16)]
                a = (plsc.load_gather(asrc_v, [sv])
                     + plsc.load_gather(adst_v, [dv]))
                a = jnp.where(a > 0, a, 0.2 * a)
                e = jnp.exp(a)
                ebuf[p][pl.ds(j * 16, 16)] = e
                off = jnp.int32(h * np_)
                idxs[p][pl.ds(j * 16, 16)] = sv + off
                idxd[p][pl.ds(j * 16, 16)] = dv + off

            def dbody(j, _):
                dv = dstv[p][pl.ds(j * 16, 16)]
                e = ebuf[p][pl.ds(j * 16, 16)]
                plsc.addupdate_scatter(den_v, [dv], e)
                return 0
            lax.fori_loop(0, b // 16, dbody, 0)

        def start_gather(p):
            pltpu.async_copy(h_hbm.at[idxs[p]], rows[p], sg[p])

        def wait_gather(p):
            pltpu.make_async_copy(h_hbm.at[idxs[p]], rows[p], sg[p]).wait()

        def mloop(p):
            @plsc.parallel_loop(0, b // 16, unroll=2)
            def mbody(j):
                ev = ebuf[p][pl.ds(j * 16, 16)]
                for l in range(16):
                    i = j * 16 + l
                    w = jnp.full((16,), ev[l], jnp.float32)
                    for kk in range(cdim // 16):
                        sl = pl.ds(kk * 16, 16)
                        rows[p][i, sl] = rows[p][i, sl] * w

        def start_scatter(p):
            pltpu.async_copy(rows[p], acc_sh.at[idxd[p]], ss[p], add=True)

        def wait_scatter(p):
            pltpu.make_async_copy(rows[p], acc_sh.at[idxd[p]], ss[p]).wait()

        def zrow(i, _):
            for kk in range(cdim // 16):
                rows[0][i, pl.ds(kk * 16, 16)] = jnp.zeros((16,), jnp.float32)
            return 0
        lax.fori_loop(0, b, zrow, 0)
        stripe0 = sid * rpt
        for ci in range(ncopies):
            start = stripe0 + jnp.minimum(ci * b, rpt - b)
            pltpu.sync_copy(rows[0], acc_sh.at[pl.ds(start, b), :])
        plsc.subcore_barrier()

        for h in range(heads):
            pltpu.sync_copy(asrcT_hbm.at[h], asrc_v)
            pltpu.sync_copy(adstT_hbm.at[h], adst_v)

            def zden(i, _):
                den_v[pl.ds(i * 16, 16)] = jnp.zeros((16,), jnp.float32)
                return 0
            lax.fori_loop(0, np_ // 16, zden, 0)

            stream_idx(0, 0)
            wait_idx(0)
            eloop(h, 0)
            start_gather(0)

            def section(k_dyn, p, has_prev, has_next):
                o = 1 - p

                @pl.when(has_next)
                def _():
                    stream_idx(k_dyn + 1, o)

                @pl.when(jnp.logical_and(has_prev, has_next))
                def _():
                    wait_scatter(o)

                @pl.when(has_next)
                def _():
                    wait_idx(o)
                    eloop(h, o)
                    start_gather(o)
                wait_gather(p)
                mloop(p)
                start_scatter(p)

            def pair(t, _):
                k0 = 2 * t
                true_ = jnp.bool_(True)
                section(k0, 0, t >= 1, true_)
                section(k0 + 1, 1, true_, t < (K // 2 - 1))
                return 0
            lax.fori_loop(0, K // 2, pair, 0)

            wait_scatter(0)
            wait_scatter(1)
            pltpu.sync_copy(den_v, den_out.at[wid, h])

        plsc.subcore_barrier()
        for ci in range(ncopies):
            start = stripe0 + jnp.minimum(ci * b, rpt - b)
            pltpu.sync_copy(acc_sh.at[pl.ds(start, b), :],
                            acc_out.at[cid, pl.ds(start, b), :])

    return edge_kernel


_edge1 = _make_edge_kernel(_H1, _C1, _B1)
_edge2 = _make_edge_kernel(1, _C2P, _B2)


def _pad_edges(edge_index, n):
    loop = jnp.arange(n, dtype=edge_index.dtype)
    npad = _EPAD - (_E + n)
    pad_ids = (jnp.arange(npad, dtype=jnp.int32) % 16) + n
    src = jnp.concatenate([edge_index[0], loop, pad_ids])
    dst = jnp.concatenate([edge_index[1], loop, pad_ids])
    return src, dst


def kernel(x, edge_index, W1, as1, ad1, b1, W2, as2, ad2, b2):
    n = x.shape[0]
    src, dst = _pad_edges(edge_index, n)
    A_s, A_d, E = _head_matrices(as1, ad1)

    h1, asrc1, adst1 = _dense1(x, W1, A_s, A_d)
    h_heads = jnp.concatenate(
        [jnp.pad(h1[:, _C1 * h:_C1 * (h + 1)], ((0, _NP - n), (0, 0)))
         for h in range(_H1)], axis=0)
    asrcT = jnp.pad(asrc1.T, ((0, 0), (0, _NP - n)))
    adstT = jnp.pad(adst1.T, ((0, 0), (0, _NP - n)))
    acc_parts, den_parts = _edge1(h_heads, asrcT, adstT, src, dst)
    acc_sum = acc_parts.sum(0)
    acc1 = jnp.concatenate(
        [acc_sum[_NP * h:_NP * h + n, :] for h in range(_H1)], axis=1)
    den1 = den_parts.sum(0)[:, :n].T

    h2, asrc2, adst2 = _dense2(acc1, den1, E, b1, W2, as2, ad2)
    h2_pad = jnp.pad(h2, ((0, _NP - n), (0, _C2P - _NCLS)))
    asrc2T = jnp.pad(asrc2.T, ((0, 0), (0, _NP - n)))
    adst2T = jnp.pad(adst2.T, ((0, 0), (0, _NP - n)))
    acc2_parts, den2_parts = _edge2(h2_pad, asrc2T, adst2T, src, dst)
    acc2 = acc2_parts.sum(0)[:n, :_NCLS]
    den2 = den2_parts.sum(0)[0, :n][:, None]

    out, z = _final(acc2, den2, b2)
    return (out, z)


# speedup vs baseline: 76.8109x; 1.2504x over previous
import functools

import jax
import jax.numpy as jnp
from jax import lax
from jax.experimental import pallas as pl
from jax.experimental.pallas import tpu as pltpu
from jax.experimental.pallas import tpu_sc as plsc

_N = 10000
_F_IN = 128
_H1, _C1 = 8, 16
_NCLS = 40
_E = 320000

_NP = 10016
_ET = 10560
_EPAD = 32 * _ET
_B1 = 480
_B2 = 480
_C2P = 48


def _head_matrices(as1, ad1):
    hc = _H1 * _C1
    j = jnp.arange(hc)
    onehot = (j[:, None] // _C1 == jnp.arange(_H1)[None, :]).astype(jnp.float32)
    A_s = onehot * as1.reshape(hc)[:, None]
    A_d = onehot * ad1.reshape(hc)[:, None]
    return A_s, A_d, onehot.T


def _dense1_body(x_ref, w_ref, As_ref, Ad_ref, h_ref, asrc_ref, adst_ref):
    h = jnp.dot(x_ref[...], w_ref[...], preferred_element_type=jnp.float32)
    h_ref[...] = h
    asrc_ref[...] = jnp.dot(h, As_ref[...], preferred_element_type=jnp.float32)
    adst_ref[...] = jnp.dot(h, Ad_ref[...], preferred_element_type=jnp.float32)


def _dense1(x, W1, A_s, A_d):
    n = x.shape[0]
    return pl.pallas_call(
        _dense1_body,
        out_shape=(
            jax.ShapeDtypeStruct((n, _H1 * _C1), jnp.float32),
            jax.ShapeDtypeStruct((n, _H1), jnp.float32),
            jax.ShapeDtypeStruct((n, _H1), jnp.float32),
        ),
    )(x, W1, A_s, A_d)


def _dense2_body(acc_ref, den_ref, E_ref, b1_ref, w2_ref, as_ref, ad_ref,
                 h2_ref, asrc_ref, adst_ref):
    rden = 1.0 / (den_ref[...] + 1e-16)
    scale = jnp.dot(rden, E_ref[...], preferred_element_type=jnp.float32)
    out1 = acc_ref[...] * scale + b1_ref[...]
    h1 = jnp.where(out1 > 0, out1, jnp.exp(jnp.minimum(out1, 0.0)) - 1.0)
    h2 = jnp.dot(h1, w2_ref[...], preferred_element_type=jnp.float32)
    h2_ref[...] = h2
    asrc_ref[...] = jnp.dot(h2, as_ref[...], preferred_element_type=jnp.float32)
    adst_ref[...] = jnp.dot(h2, ad_ref[...], preferred_element_type=jnp.float32)


def _dense2(acc1, den1, E, b1, W2, as2, ad2):
    n = acc1.shape[0]
    return pl.pallas_call(
        _dense2_body,
        out_shape=(
            jax.ShapeDtypeStruct((n, _NCLS), jnp.float32),
            jax.ShapeDtypeStruct((n, 1), jnp.float32),
            jax.ShapeDtypeStruct((n, 1), jnp.float32),
        ),
    )(acc1, den1, E, b1, W2, as2.T, ad2.T)


def _final_body(acc_ref, den_ref, b2_ref, out_ref, z_ref):
    z = acc_ref[...] / (den_ref[...] + 1e-16) + b2_ref[...]
    z_ref[...] = z
    m = jnp.max(z, axis=-1, keepdims=True)
    s = jnp.log(jnp.sum(jnp.exp(z - m), axis=-1, keepdims=True))
    out_ref[...] = z - m - s


def _final(acc2, den2, b2):
    n = acc2.shape[0]
    return pl.pallas_call(
        _final_body,
        out_shape=(
            jax.ShapeDtypeStruct((n, _NCLS), jnp.float32),
            jax.ShapeDtypeStruct((n, _NCLS), jnp.float32),
        ),
    )(acc2, den2, b2)


def _make_edge_kernel(heads, cdim, b):
    np_ = _NP
    K = _ET // b
    assert K % 2 == 0
    rows_total = heads * np_
    rpt = rows_total // 16
    ncopies = -(-rpt // b)

    mesh = plsc.VectorSubcoreMesh(core_axis_name="c", subcore_axis_name="s")

    edge_bufs = [
        pltpu.VMEM((b,), jnp.int32),
        pltpu.VMEM((b,), jnp.int32),
        pltpu.VMEM((b,), jnp.int32),
        pltpu.VMEM((b,), jnp.int32),
        pltpu.VMEM((b,), jnp.float32),
        pltpu.VMEM((b, cdim), jnp.float32),
    ]

    @functools.partial(
        pl.kernel,
        out_type=(
            jax.ShapeDtypeStruct((2, np_, heads * cdim), jnp.float32),
            jax.ShapeDtypeStruct((32, heads, np_), jnp.float32),
        ),
        mesh=mesh,
        compiler_params=pltpu.CompilerParams(
            needs_layout_passes=False, use_tc_tiling_on_sc=False),
        scratch_types=[
            pltpu.VMEM((np_,), jnp.float32),
            pltpu.VMEM((np_,), jnp.float32),
            pltpu.VMEM((np_,), jnp.float32),
            pltpu.VMEM_SHARED((rows_total, cdim), jnp.float32),
        ] + edge_bufs + edge_bufs + [pltpu.SemaphoreType.DMA] * 8,
    )
    def edge_kernel(h_hbm, asrcT_hbm, adstT_hbm, src_hbm, dst_hbm,
                    acc_out, den_out,
                    asrc_v, adst_v, den_v, acc_sh, *bufs_and_sems):
        srcv = (bufs_and_sems[0], bufs_and_sems[6])
        dstv = (bufs_and_sems[1], bufs_and_sems[7])
        idxs = (bufs_and_sems[2], bufs_and_sems[8])
        idxd = (bufs_and_sems[3], bufs_and_sems[9])
        ebuf = (bufs_and_sems[4], bufs_and_sems[10])
        rows = (bufs_and_sems[5], bufs_and_sems[11])
        sis = (bufs_and_sems[12], bufs_and_sems[13])
        sid_ = (bufs_and_sems[14], bufs_and_sems[15])
        sg = (bufs_and_sems[16], bufs_and_sems[17])
        ss = (bufs_and_sems[18], bufs_and_sems[19])

        cid = lax.axis_index("c")
        sid = lax.axis_index("s")
        wid = cid * 16 + sid

        def stream_idx(k, p):
            base = wid * _ET + k * b
            pltpu.async_copy(src_hbm.at[pl.ds(base, b)], srcv[p], sis[p])
            pltpu.async_copy(dst_hbm.at[pl.ds(base, b)], dstv[p], sid_[p])

        def wait_idx(p):
            pltpu.make_async_copy(src_hbm.at[pl.ds(0, b)], srcv[p], sis[p]).wait()
            pltpu.make_async_copy(dst_hbm.at[pl.ds(0, b)], dstv[p], sid_[p]).wait()

        def eloop(h, p):
            @plsc.parallel_loop(0, b // 16, unroll=2)
            def ebody(j):
                sv = srcv[p][pl.ds(j * 16, 16)]
                dv = dstv[p][pl.ds(j * 16, 16)]
                a = (plsc.load_gather(asrc_v, [sv])
                     + plsc.load_gather(adst_v, [dv]))
                a = jnp.where(a > 0, a, 0.2 * a)
                e = jnp.exp(a)
                ebuf[p][pl.ds(j * 16, 16)] = e
                off = jnp.int32(h * np_)
                idxs[p][pl.ds(j * 16, 16)] = sv + off
                idxd[p][pl.ds(j * 16, 16)] = dv + off

            def dbody(j, _):
                dv = dstv[p][pl.ds(j * 16, 16)]
                e = ebuf[p][pl.ds(j * 16, 16)]
                plsc.addupdate_scatter(den_v, [dv], e)
                return 0
            lax.fori_loop(0, b // 16, dbody, 0)

        def start_gather(p):
            pltpu.async_copy(h_hbm.at[idxs[p]], rows[p], sg[p])

        def wait_gather(p):
            pltpu.make_async_copy(h_hbm.at[idxs[p]], rows[p], sg[p]).wait()

        def mloop(p):
            @plsc.parallel_loop(0, b // 16, unroll=2)
            def mbody(j):
                ev = ebuf[p][pl.ds(j * 16, 16)]
                for l in range(16):
                    i = j * 16 + l
                    w = jnp.full((16,), ev[l], jnp.float32)
                    for kk in range(cdim // 16):
                        sl = pl.ds(kk * 16, 16)
                        rows[p][i, sl] = rows[p][i, sl] * w

        def start_scatter(p):
            pltpu.async_copy(rows[p], acc_sh.at[idxd[p]], ss[p], add=True)

        def wait_scatter(p):
            pltpu.make_async_copy(rows[p], acc_sh.at[idxd[p]], ss[p]).wait()

        def zrow(i, _):
            for kk in range(cdim // 16):
                rows[0][i, pl.ds(kk * 16, 16)] = jnp.zeros((16,), jnp.float32)
            return 0
        lax.fori_loop(0, b, zrow, 0)
        stripe0 = sid * rpt
        for ci in range(ncopies):
            start = stripe0 + jnp.minimum(ci * b, rpt - b)
            pltpu.sync_copy(rows[0], acc_sh.at[pl.ds(start, b), :])
        plsc.subcore_barrier()

        for h in range(heads):
            pltpu.sync_copy(asrcT_hbm.at[h], asrc_v)
            pltpu.sync_copy(adstT_hbm.at[h], adst_v)

            def zden(i, _):
                den_v[pl.ds(i * 16, 16)] = jnp.zeros((16,), jnp.float32)
                return 0
            lax.fori_loop(0, np_ // 16, zden, 0)

            stream_idx(0, 0)
            wait_idx(0)
            eloop(h, 0)
            start_gather(0)

            def section(k_dyn, p, has_prev, has_next):
                o = 1 - p

                @pl.when(has_next)
                def _():
                    stream_idx(k_dyn + 1, o)

                @pl.when(jnp.logical_and(has_prev, has_next))
                def _():
                    wait_scatter(o)

                @pl.when(has_next)
                def _():
                    wait_idx(o)
                    eloop(h, o)
                    start_gather(o)
                wait_gather(p)
                mloop(p)
                start_scatter(p)

            def pair(t, _):
                k0 = 2 * t
                true_ = jnp.bool_(True)
                section(k0, 0, t >= 1, true_)
                section(k0 + 1, 1, true_, t < (K // 2 - 1))
                return 0
            lax.fori_loop(0, K // 2, pair, 0)

            wait_scatter(0)
            wait_scatter(1)
            pltpu.sync_copy(den_v, den_out.at[wid, h])

        plsc.subcore_barrier()
        npt = np_ // 16
        for h in range(heads):
            pltpu.sync_copy(
                acc_sh.at[pl.ds(h * np_ + sid * npt, npt), :],
                acc_out.at[cid, pl.ds(sid * npt, npt), pl.ds(h * cdim, cdim)])

    return edge_kernel


_edge1 = _make_edge_kernel(_H1, _C1, _B1)
_edge2 = _make_edge_kernel(1, _C2P, _B2)


def _pad_edges(edge_index, n):
    loop = jnp.arange(n, dtype=edge_index.dtype)
    npad = _EPAD - (_E + n)
    pad_ids = (jnp.arange(npad, dtype=jnp.int32) % 16) + n
    src = jnp.concatenate([edge_index[0], loop, pad_ids])
    dst = jnp.concatenate([edge_index[1], loop, pad_ids])
    return src, dst


def kernel(x, edge_index, W1, as1, ad1, b1, W2, as2, ad2, b2):
    n = x.shape[0]
    src, dst = _pad_edges(edge_index, n)
    A_s, A_d, E = _head_matrices(as1, ad1)

    h1, asrc1, adst1 = _dense1(x, W1, A_s, A_d)
    h_heads = jnp.concatenate(
        [jnp.pad(h1[:, _C1 * h:_C1 * (h + 1)], ((0, _NP - n), (0, 0)))
         for h in range(_H1)], axis=0)
    asrcT = jnp.pad(asrc1.T, ((0, 0), (0, _NP - n)))
    adstT = jnp.pad(adst1.T, ((0, 0), (0, _NP - n)))
    acc_parts, den_parts = _edge1(h_heads, asrcT, adstT, src, dst)
    acc1 = acc_parts.sum(0)[:n]
    den1 = den_parts.sum(0)[:, :n].T

    h2, asrc2, adst2 = _dense2(acc1, den1, E, b1, W2, as2, ad2)
    h2_pad = jnp.pad(h2, ((0, _NP - n), (0, _C2P - _NCLS)))
    asrc2T = jnp.pad(asrc2.T, ((0, 0), (0, _NP - n)))
    adst2T = jnp.pad(adst2.T, ((0, 0), (0, _NP - n)))
    acc2_parts, den2_parts = _edge2(h2_pad, asrc2T, adst2T, src, dst)
    acc2 = acc2_parts.sum(0)[:n, :_NCLS]
    den2 = den2_parts.sum(0)[0, :n][:, None]

    out, z = _final(acc2, den2, b2)
    return (out, z)


# speedup vs baseline: 80.3509x; 1.0461x over previous
import functools

import jax
import jax.numpy as jnp
from jax import lax
from jax.experimental import pallas as pl
from jax.experimental.pallas import tpu as pltpu
from jax.experimental.pallas import tpu_sc as plsc

_N = 10000
_F_IN = 128
_H1, _C1 = 8, 16
_NCLS = 40
_E = 320000

_NP = 10016
_ET = 10560
_EPAD = 32 * _ET
_B1 = 480
_B2 = 480
_C2P = 48


def _expander():
    j = jnp.arange(_H1 * _C1)
    return (j[None, :] // _C1 == jnp.arange(_H1)[:, None]).astype(jnp.float32)


def _dense1_body(x_ref, w_ref, as_ref, ad_ref, hh_ref, asrc_ref, adst_ref):
    n = x_ref.shape[0]
    hb = jnp.dot(x_ref[...], w_ref[0], preferred_element_type=jnp.float32)
    hh_ref[:n, :] = hb
    hh_ref[n:, :] = jnp.zeros((_NP - _N, _C1), jnp.float32)
    dn = (((1,), (1,)), ((), ()))
    asrc_ref[0, :, :n] = jax.lax.dot_general(
        as_ref[0], hb, dn, preferred_element_type=jnp.float32)
    asrc_ref[0, :, n:] = jnp.zeros((1, _NP - _N), jnp.float32)
    adst_ref[0, :, :n] = jax.lax.dot_general(
        ad_ref[0], hb, dn, preferred_element_type=jnp.float32)
    adst_ref[0, :, n:] = jnp.zeros((1, _NP - _N), jnp.float32)


def _dense1(x, W1, as1, ad1):
    n = x.shape[0]
    W1h = W1.reshape(_F_IN, _H1, _C1).transpose(1, 0, 2)
    hh, asrcT, adstT = pl.pallas_call(
        _dense1_body,
        grid=(_H1,),
        in_specs=[
            pl.BlockSpec((n, _F_IN), lambda h: (0, 0)),
            pl.BlockSpec((1, _F_IN, _C1), lambda h: (h, 0, 0)),
            pl.BlockSpec((1, 1, _C1), lambda h: (h, 0, 0)),
            pl.BlockSpec((1, 1, _C1), lambda h: (h, 0, 0)),
        ],
        out_specs=(
            pl.BlockSpec((_NP, _C1), lambda h: (h, 0)),
            pl.BlockSpec((1, 1, _NP), lambda h: (h, 0, 0)),
            pl.BlockSpec((1, 1, _NP), lambda h: (h, 0, 0)),
        ),
        out_shape=(
            jax.ShapeDtypeStruct((_H1 * _NP, _C1), jnp.float32),
            jax.ShapeDtypeStruct((_H1, 1, _NP), jnp.float32),
            jax.ShapeDtypeStruct((_H1, 1, _NP), jnp.float32),
        ),
    )(x, W1h, as1.reshape(_H1, 1, _C1), ad1.reshape(_H1, 1, _C1))
    return hh, asrcT.reshape(_H1, _NP), adstT.reshape(_H1, _NP)


def _dense2_body(acc_ref, den_ref, E_ref, b1_ref, w2_ref, as_ref, ad_ref,
                 h2_ref, asrc_ref, adst_ref):
    rden = 1.0 / (den_ref[...] + 1e-16)
    scale = jnp.dot(rden, E_ref[...], preferred_element_type=jnp.float32)
    out1 = acc_ref[...] * scale + b1_ref[...]
    h1 = jnp.where(out1 > 0, out1, jnp.exp(jnp.minimum(out1, 0.0)) - 1.0)
    h2 = jnp.dot(h1, w2_ref[...], preferred_element_type=jnp.float32)
    h2_ref[...] = h2
    asrc_ref[...] = jnp.dot(h2, as_ref[...], preferred_element_type=jnp.float32)
    adst_ref[...] = jnp.dot(h2, ad_ref[...], preferred_element_type=jnp.float32)


def _dense2(acc1, den1, E, b1, W2, as2, ad2):
    n = acc1.shape[0]
    return pl.pallas_call(
        _dense2_body,
        out_shape=(
            jax.ShapeDtypeStruct((n, _NCLS), jnp.float32),
            jax.ShapeDtypeStruct((n, 1), jnp.float32),
            jax.ShapeDtypeStruct((n, 1), jnp.float32),
        ),
    )(acc1, den1, E, b1, W2, as2.T, ad2.T)


def _final_body(acc_ref, den_ref, b2_ref, out_ref, z_ref):
    z = acc_ref[...] / (den_ref[...] + 1e-16) + b2_ref[...]
    z_ref[...] = z
    m = jnp.max(z, axis=-1, keepdims=True)
    s = jnp.log(jnp.sum(jnp.exp(z - m), axis=-1, keepdims=True))
    out_ref[...] = z - m - s


def _final(acc2, den2, b2):
    n = acc2.shape[0]
    return pl.pallas_call(
        _final_body,
        out_shape=(
            jax.ShapeDtypeStruct((n, _NCLS), jnp.float32),
            jax.ShapeDtypeStruct((n, _NCLS), jnp.float32),
        ),
    )(acc2, den2, b2)


def _make_edge_kernel(heads, cdim, b):
    np_ = _NP
    K = _ET // b
    assert K % 2 == 0
    rows_total = heads * np_
    rpt = rows_total // 16
    ncopies = -(-rpt // b)

    mesh = plsc.VectorSubcoreMesh(core_axis_name="c", subcore_axis_name="s")

    edge_bufs = [
        pltpu.VMEM((b,), jnp.int32),
        pltpu.VMEM((b,), jnp.int32),
        pltpu.VMEM((b,), jnp.int32),
        pltpu.VMEM((b,), jnp.int32),
        pltpu.VMEM((b,), jnp.float32),
        pltpu.VMEM((b, cdim), jnp.float32),
    ]

    @functools.partial(
        pl.kernel,
        out_type=(
            jax.ShapeDtypeStruct((2, np_, heads * cdim), jnp.float32),
            jax.ShapeDtypeStruct((32, heads, np_), jnp.float32),
        ),
        mesh=mesh,
        compiler_params=pltpu.CompilerParams(
            needs_layout_passes=False, use_tc_tiling_on_sc=False),
        scratch_types=[
            pltpu.VMEM((np_,), jnp.float32),
            pltpu.VMEM((np_,), jnp.float32),
            pltpu.VMEM((np_,), jnp.float32),
            pltpu.VMEM_SHARED((rows_total, cdim), jnp.float32),
        ] + edge_bufs + edge_bufs + [pltpu.SemaphoreType.DMA] * 8,
    )
    def edge_kernel(h_hbm, asrcT_hbm, adstT_hbm, src_hbm, dst_hbm,
                    acc_out, den_out,
                    asrc_v, adst_v, den_v, acc_sh, *bufs_and_sems):
        srcv = (bufs_and_sems[0], bufs_and_sems[6])
        dstv = (bufs_and_sems[1], bufs_and_sems[7])
        idxs = (bufs_and_sems[2], bufs_and_sems[8])
        idxd = (bufs_and_sems[3], bufs_and_sems[9])
        ebuf = (bufs_and_sems[4], bufs_and_sems[10])
        rows = (bufs_and_sems[5], bufs_and_sems[11])
        sis = (bufs_and_sems[12], bufs_and_sems[13])
        sid_ = (bufs_and_sems[14], bufs_and_sems[15])
        sg = (bufs_and_sems[16], bufs_and_sems[17])
        ss = (bufs_and_sems[18], bufs_and_sems[19])

        cid = lax.axis_index("c")
        sid = lax.axis_index("s")
        wid = cid * 16 + sid

        def stream_idx(k, p):
            base = wid * _ET + k * b
            pltpu.async_copy(src_hbm.at[pl.ds(base, b)], srcv[p], sis[p])
            pltpu.async_copy(dst_hbm.at[pl.ds(base, b)], dstv[p], sid_[p])

        def wait_idx(p):
            pltpu.make_async_copy(src_hbm.at[pl.ds(0, b)], srcv[p], sis[p]).wait()
            pltpu.make_async_copy(dst_hbm.at[pl.ds(0, b)], dstv[p], sid_[p]).wait()

        def eloop(h, p):
            @plsc.parallel_loop(0, b // 16, unroll=2)
            def ebody(j):
                sv = srcv[p][pl.ds(j * 16, 16)]
                dv = dstv[p][pl.ds(j * 16, 16)]
                a = (plsc.load_gather(asrc_v, [sv])
                     + plsc.load_gather(adst_v, [dv]))
                a = jnp.where(a > 0, a, 0.2 * a)
                e = jnp.exp(a)
                ebuf[p][pl.ds(j * 16, 16)] = e
                off = jnp.int32(h * np_)
                idxs[p][pl.ds(j * 16, 16)] = sv + off
                idxd[p][pl.ds(j * 16, 16)] = dv + off

            def dbody(j, _):
                dv = dstv[p][pl.ds(j * 16, 16)]
                e = ebuf[p][pl.ds(j * 16, 16)]
                plsc.addupdate_scatter(den_v, [dv], e)
                return 0
            lax.fori_loop(0, b // 16, dbody, 0)

        def start_gather(p):
            pltpu.async_copy(h_hbm.at[idxs[p]], rows[p], sg[p])

        def wait_gather(p):
            pltpu.make_async_copy(h_hbm.at[idxs[p]], rows[p], sg[p]).wait()

        def mloop(p):
            @plsc.parallel_loop(0, b // 16, unroll=2)
            def mbody(j):
                ev = ebuf[p][pl.ds(j * 16, 16)]
                for l in range(16):
                    i = j * 16 + l
                    w = jnp.full((16,), ev[l], jnp.float32)
                    for kk in range(cdim // 16):
                        sl = pl.ds(kk * 16, 16)
                        rows[p][i, sl] = rows[p][i, sl] * w

        def start_scatter(p):
            pltpu.async_copy(rows[p], acc_sh.at[idxd[p]], ss[p], add=True)

        def wait_scatter(p):
            pltpu.make_async_copy(rows[p], acc_sh.at[idxd[p]], ss[p]).wait()

        def zrow(i, _):
            for kk in range(cdim // 16):
                rows[0][i, pl.ds(kk * 16, 16)] = jnp.zeros((16,), jnp.float32)
            return 0
        lax.fori_loop(0, b, zrow, 0)
        stripe0 = sid * rpt
        for ci in range(ncopies):
            start = stripe0 + jnp.minimum(ci * b, rpt - b)
            pltpu.sync_copy(rows[0], acc_sh.at[pl.ds(start, b), :])
        plsc.subcore_barrier()

        for h in range(heads):
            pltpu.sync_copy(asrcT_hbm.at[h], asrc_v)
            pltpu.sync_copy(adstT_hbm.at[h], adst_v)

            def zden(i, _):
                den_v[pl.ds(i * 16, 16)] = jnp.zeros((16,), jnp.float32)
                return 0
            lax.fori_loop(0, np_ // 16, zden, 0)

            stream_idx(0, 0)
            wait_idx(0)
            eloop(h, 0)
            start_gather(0)

            def section(k_dyn, p, has_prev, has_next):
                o = 1 - p

                @pl.when(has_next)
                def _():
                    stream_idx(k_dyn + 1, o)

                @pl.when(jnp.logical_and(has_prev, has_next))
                def _():
                    wait_scatter(o)

                @pl.when(has_next)
                def _():
                    wait_idx(o)
                    eloop(h, o)
                    start_gather(o)
                wait_gather(p)
                mloop(p)
                start_scatter(p)

            def pair(t, _):
                k0 = 2 * t
                true_ = jnp.bool_(True)
                section(k0, 0, t >= 1, true_)
                section(k0 + 1, 1, true_, t < (K // 2 - 1))
                return 0
            lax.fori_loop(0, K // 2, pair, 0)

            wait_scatter(0)
            wait_scatter(1)
            pltpu.sync_copy(den_v, den_out.at[wid, h])

        plsc.subcore_barrier()
        npt = np_ // 16
        for h in range(heads):
            pltpu.sync_copy(
                acc_sh.at[pl.ds(h * np_ + sid * npt, npt), :],
                acc_out.at[cid, pl.ds(sid * npt, npt), pl.ds(h * cdim, cdim)])

    return edge_kernel


_edge1 = _make_edge_kernel(_H1, _C1, _B1)
_edge2 = _make_edge_kernel(1, _C2P, _B2)


def _pad_edges(edge_index, n):
    loop = jnp.arange(n, dtype=edge_index.dtype)
    npad = _EPAD - (_E + n)
    pad_ids = (jnp.arange(npad, dtype=jnp.int32) % 16) + n
    src = jnp.concatenate([edge_index[0], loop, pad_ids])
    dst = jnp.concatenate([edge_index[1], loop, pad_ids])
    return src, dst


def kernel(x, edge_index, W1, as1, ad1, b1, W2, as2, ad2, b2):
    n = x.shape[0]
    src, dst = _pad_edges(edge_index, n)
    E = _expander()

    h_heads, asrcT, adstT = _dense1(x, W1, as1, ad1)
    acc_parts, den_parts = _edge1(h_heads, asrcT, adstT, src, dst)
    acc1 = acc_parts.sum(0)[:n]
    den1 = den_parts.sum(0)[:, :n].T

    h2, asrc2, adst2 = _dense2(acc1, den1, E, b1, W2, as2, ad2)
    h2_pad = jnp.pad(h2, ((0, _NP - n), (0, _C2P - _NCLS)))
    asrc2T = jnp.pad(asrc2.T, ((0, 0), (0, _NP - n)))
    adst2T = jnp.pad(adst2.T, ((0, 0), (0, _NP - n)))
    acc2_parts, den2_parts = _edge2(h2_pad, asrc2T, adst2T, src, dst)
    acc2 = acc2_parts.sum(0)[:n, :_NCLS]
    den2 = den2_parts.sum(0)[0, :n][:, None]

    out, z = _final(acc2, den2, b2)
    return (out, z)


# speedup vs baseline: 82.0024x; 1.0206x over previous
import functools

import jax
import jax.numpy as jnp
from jax import lax
from jax.experimental import pallas as pl
from jax.experimental.pallas import tpu as pltpu
from jax.experimental.pallas import tpu_sc as plsc

_N = 10000
_F_IN = 128
_H1, _C1 = 8, 16
_NCLS = 40
_E = 320000

_NP = 10016
_ET = 10560
_EPAD = 32 * _ET
_B1 = 480
_B2 = 480
_C2P = 48


def _expander():
    j = jnp.arange(_H1 * _C1)
    return (j[None, :] // _C1 == jnp.arange(_H1)[:, None]).astype(jnp.float32)


def _dense1_body(x_ref, w_ref, as_ref, ad_ref, hh_ref, asrc_ref, adst_ref):
    n = x_ref.shape[0]
    hb = jnp.dot(x_ref[...], w_ref[0], preferred_element_type=jnp.float32)
    hh_ref[:n, :] = hb
    hh_ref[n:, :] = jnp.zeros((_NP - _N, _C1), jnp.float32)
    dn = (((1,), (1,)), ((), ()))
    asrc_ref[0, :, :n] = jax.lax.dot_general(
        as_ref[0], hb, dn, preferred_element_type=jnp.float32)
    asrc_ref[0, :, n:] = jnp.zeros((1, _NP - _N), jnp.float32)
    adst_ref[0, :, :n] = jax.lax.dot_general(
        ad_ref[0], hb, dn, preferred_element_type=jnp.float32)
    adst_ref[0, :, n:] = jnp.zeros((1, _NP - _N), jnp.float32)


def _dense1(x, W1, as1, ad1):
    n = x.shape[0]
    W1h = W1.reshape(_F_IN, _H1, _C1).transpose(1, 0, 2)
    hh, asrcT, adstT = pl.pallas_call(
        _dense1_body,
        grid=(_H1,),
        in_specs=[
            pl.BlockSpec((n, _F_IN), lambda h: (0, 0)),
            pl.BlockSpec((1, _F_IN, _C1), lambda h: (h, 0, 0)),
            pl.BlockSpec((1, 1, _C1), lambda h: (h, 0, 0)),
            pl.BlockSpec((1, 1, _C1), lambda h: (h, 0, 0)),
        ],
        out_specs=(
            pl.BlockSpec((_NP, _C1), lambda h: (h, 0)),
            pl.BlockSpec((1, 1, _NP), lambda h: (h, 0, 0)),
            pl.BlockSpec((1, 1, _NP), lambda h: (h, 0, 0)),
        ),
        out_shape=(
            jax.ShapeDtypeStruct((_H1 * _NP, _C1), jnp.float32),
            jax.ShapeDtypeStruct((_H1, 1, _NP), jnp.float32),
            jax.ShapeDtypeStruct((_H1, 1, _NP), jnp.float32),
        ),
    )(x, W1h, as1.reshape(_H1, 1, _C1), ad1.reshape(_H1, 1, _C1))
    return hh, asrcT.reshape(_H1, _NP), adstT.reshape(_H1, _NP)


def _dense2_body(acc_ref, den_ref, E_ref, b1_ref, w2_ref, as_ref, ad_ref,
                 h2_ref, asrc_ref, adst_ref):
    rden = 1.0 / (den_ref[...] + 1e-16)
    scale = jnp.dot(rden, E_ref[...], preferred_element_type=jnp.float32)
    acc = acc_ref[0, :_N, :] + acc_ref[1, :_N, :]
    out1 = acc * scale + b1_ref[...]
    h1 = jnp.where(out1 > 0, out1, jnp.exp(jnp.minimum(out1, 0.0)) - 1.0)
    h2 = jnp.dot(h1, w2_ref[...], preferred_element_type=jnp.float32)
    h2_ref[...] = h2
    asrc_ref[...] = jnp.dot(h2, as_ref[...], preferred_element_type=jnp.float32)
    adst_ref[...] = jnp.dot(h2, ad_ref[...], preferred_element_type=jnp.float32)


def _dense2(acc_parts, den1, E, b1, W2, as2, ad2):
    n = _N
    return pl.pallas_call(
        _dense2_body,
        out_shape=(
            jax.ShapeDtypeStruct((n, _NCLS), jnp.float32),
            jax.ShapeDtypeStruct((n, 1), jnp.float32),
            jax.ShapeDtypeStruct((n, 1), jnp.float32),
        ),
    )(acc_parts, den1, E, b1, W2, as2.T, ad2.T)


def _final_body(acc_ref, den_ref, b2_ref, out_ref, z_ref):
    acc = acc_ref[0, :_N, :_NCLS] + acc_ref[1, :_N, :_NCLS]
    z = acc / (den_ref[...] + 1e-16) + b2_ref[...]
    z_ref[...] = z
    m = jnp.max(z, axis=-1, keepdims=True)
    s = jnp.log(jnp.sum(jnp.exp(z - m), axis=-1, keepdims=True))
    out_ref[...] = z - m - s


def _final(acc2_parts, den2, b2):
    n = _N
    return pl.pallas_call(
        _final_body,
        out_shape=(
            jax.ShapeDtypeStruct((n, _NCLS), jnp.float32),
            jax.ShapeDtypeStruct((n, _NCLS), jnp.float32),
        ),
    )(acc2_parts, den2, b2)


def _make_edge_kernel(heads, cdim, b):
    np_ = _NP
    K = _ET // b
    assert K % 2 == 0
    rows_total = heads * np_
    rpt = rows_total // 16
    ncopies = -(-rpt // b)

    mesh = plsc.VectorSubcoreMesh(core_axis_name="c", subcore_axis_name="s")

    edge_bufs = [
        pltpu.VMEM((b,), jnp.int32),
        pltpu.VMEM((b,), jnp.int32),
        pltpu.VMEM((b,), jnp.int32),
        pltpu.VMEM((b,), jnp.int32),
        pltpu.VMEM((b,), jnp.float32),
        pltpu.VMEM((b, cdim), jnp.float32),
    ]

    @functools.partial(
        pl.kernel,
        out_type=(
            jax.ShapeDtypeStruct((2, np_, heads * cdim), jnp.float32),
            jax.ShapeDtypeStruct((32, heads, np_), jnp.float32),
        ),
        mesh=mesh,
        compiler_params=pltpu.CompilerParams(
            needs_layout_passes=False, use_tc_tiling_on_sc=False),
        scratch_types=[
            pltpu.VMEM((np_,), jnp.float32),
            pltpu.VMEM((np_,), jnp.float32),
            pltpu.VMEM((np_,), jnp.float32),
            pltpu.VMEM_SHARED((rows_total, cdim), jnp.float32),
        ] + edge_bufs + edge_bufs + [pltpu.SemaphoreType.DMA] * 8,
    )
    def edge_kernel(h_hbm, asrcT_hbm, adstT_hbm, src_hbm, dst_hbm,
                    acc_out, den_out,
                    asrc_v, adst_v, den_v, acc_sh, *bufs_and_sems):
        srcv = (bufs_and_sems[0], bufs_and_sems[6])
        dstv = (bufs_and_sems[1], bufs_and_sems[7])
        idxs = (bufs_and_sems[2], bufs_and_sems[8])
        idxd = (bufs_and_sems[3], bufs_and_sems[9])
        ebuf = (bufs_and_sems[4], bufs_and_sems[10])
        rows = (bufs_and_sems[5], bufs_and_sems[11])
        sis = (bufs_and_sems[12], bufs_and_sems[13])
        sid_ = (bufs_and_sems[14], bufs_and_sems[15])
        sg = (bufs_and_sems[16], bufs_and_sems[17])
        ss = (bufs_and_sems[18], bufs_and_sems[19])

        cid = lax.axis_index("c")
        sid = lax.axis_index("s")
        wid = cid * 16 + sid

        def stream_idx(k, p):
            base = wid * _ET + k * b
            pltpu.async_copy(src_hbm.at[pl.ds(base, b)], srcv[p], sis[p])
            pltpu.async_copy(dst_hbm.at[pl.ds(base, b)], dstv[p], sid_[p])

        def wait_idx(p):
            pltpu.make_async_copy(src_hbm.at[pl.ds(0, b)], srcv[p], sis[p]).wait()
            pltpu.make_async_copy(dst_hbm.at[pl.ds(0, b)], dstv[p], sid_[p]).wait()

        def eloop(h, p):
            @plsc.parallel_loop(0, b // 16, unroll=2)
            def ebody(j):
                sv = srcv[p][pl.ds(j * 16, 16)]
                dv = dstv[p][pl.ds(j * 16, 16)]
                a = (plsc.load_gather(asrc_v, [sv])
                     + plsc.load_gather(adst_v, [dv]))
                a = jnp.where(a > 0, a, 0.2 * a)
                e = jnp.exp(a)
                ebuf[p][pl.ds(j * 16, 16)] = e
                off = jnp.int32(h * np_)
                idxs[p][pl.ds(j * 16, 16)] = sv + off
                idxd[p][pl.ds(j * 16, 16)] = dv + off

            def dbody(j, _):
                dv = dstv[p][pl.ds(j * 16, 16)]
                e = ebuf[p][pl.ds(j * 16, 16)]
                plsc.addupdate_scatter(den_v, [dv], e)
                return 0
            lax.fori_loop(0, b // 16, dbody, 0)

        def start_gather(p):
            pltpu.async_copy(h_hbm.at[idxs[p]], rows[p], sg[p])

        def wait_gather(p):
            pltpu.make_async_copy(h_hbm.at[idxs[p]], rows[p], sg[p]).wait()

        def mloop(p):
            @plsc.parallel_loop(0, b // 16, unroll=2)
            def mbody(j):
                ev = ebuf[p][pl.ds(j * 16, 16)]
                for l in range(16):
                    i = j * 16 + l
                    w = jnp.full((16,), ev[l], jnp.float32)
                    for kk in range(cdim // 16):
                        sl = pl.ds(kk * 16, 16)
                        rows[p][i, sl] = rows[p][i, sl] * w

        def start_scatter(p):
            pltpu.async_copy(rows[p], acc_sh.at[idxd[p]], ss[p], add=True)

        def wait_scatter(p):
            pltpu.make_async_copy(rows[p], acc_sh.at[idxd[p]], ss[p]).wait()

        def zrow(i, _):
            for kk in range(cdim // 16):
                rows[0][i, pl.ds(kk * 16, 16)] = jnp.zeros((16,), jnp.float32)
            return 0
        lax.fori_loop(0, b, zrow, 0)
        stripe0 = sid * rpt
        for ci in range(ncopies):
            start = stripe0 + jnp.minimum(ci * b, rpt - b)
            pltpu.sync_copy(rows[0], acc_sh.at[pl.ds(start, b), :])
        plsc.subcore_barrier()

        for h in range(heads):
            pltpu.sync_copy(asrcT_hbm.at[h], asrc_v)
            pltpu.sync_copy(adstT_hbm.at[h], adst_v)

            def zden(i, _):
                den_v[pl.ds(i * 16, 16)] = jnp.zeros((16,), jnp.float32)
                return 0
            lax.fori_loop(0, np_ // 16, zden, 0)

            stream_idx(0, 0)
            wait_idx(0)
            eloop(h, 0)
            start_gather(0)

            def section(k_dyn, p, has_prev, has_next):
                o = 1 - p

                @pl.when(has_next)
                def _():
                    stream_idx(k_dyn + 1, o)

                @pl.when(jnp.logical_and(has_prev, has_next))
                def _():
                    wait_scatter(o)

                @pl.when(has_next)
                def _():
                    wait_idx(o)
                    eloop(h, o)
                    start_gather(o)
                wait_gather(p)
                mloop(p)
                start_scatter(p)

            def pair(t, _):
                k0 = 2 * t
                true_ = jnp.bool_(True)
                section(k0, 0, t >= 1, true_)
                section(k0 + 1, 1, true_, t < (K // 2 - 1))
                return 0
            lax.fori_loop(0, K // 2, pair, 0)

            wait_scatter(0)
            wait_scatter(1)
            pltpu.sync_copy(den_v, den_out.at[wid, h])

        plsc.subcore_barrier()
        npt = np_ // 16
        for h in range(heads):
            pltpu.sync_copy(
                acc_sh.at[pl.ds(h * np_ + sid * npt, npt), :],
                acc_out.at[cid, pl.ds(sid * npt, npt), pl.ds(h * cdim, cdim)])

    return edge_kernel


_edge1 = _make_edge_kernel(_H1, _C1, _B1)
_edge2 = _make_edge_kernel(1, _C2P, _B2)


def _pad_edges(edge_index, n):
    loop = jnp.arange(n, dtype=edge_index.dtype)
    npad = _EPAD - (_E + n)
    pad_ids = (jnp.arange(npad, dtype=jnp.int32) % 16) + n
    src = jnp.concatenate([edge_index[0], loop, pad_ids])
    dst = jnp.concatenate([edge_index[1], loop, pad_ids])
    return src, dst


def kernel(x, edge_index, W1, as1, ad1, b1, W2, as2, ad2, b2):
    n = x.shape[0]
    src, dst = _pad_edges(edge_index, n)
    E = _expander()

    h_heads, asrcT, adstT = _dense1(x, W1, as1, ad1)
    acc_parts, den_parts = _edge1(h_heads, asrcT, adstT, src, dst)
    den1 = den_parts.sum(0)[:, :n].T

    h2, asrc2, adst2 = _dense2(acc_parts, den1, E, b1, W2, as2, ad2)
    h2_pad = jnp.pad(h2, ((0, _NP - n), (0, _C2P - _NCLS)))
    asrc2T = jnp.pad(asrc2.T, ((0, 0), (0, _NP - n)))
    adst2T = jnp.pad(adst2.T, ((0, 0), (0, _NP - n)))
    acc2_parts, den2_parts = _edge2(h2_pad, asrc2T, adst2T, src, dst)
    den2 = den2_parts.sum(0)[0, :n][:, None]

    out, z = _final(acc2_parts, den2, b2)
    return (out, z)
